# Initial kernel scaffold; baseline (speedup 1.0000x reference)
import jax
import jax.numpy as jnp
from jax.experimental import pallas as pl


def kernel(x, edge_index, edge_attr, emb_table, atom_w, atom_b, ee_w, ee_b, ew1, eb1, ew2, eb2, nw1, nb1, nw2, nb2, ow1, ob1, ow2, ob2):
    raise NotImplementedError("write your pallas kernel here")


# speedup vs baseline: 3.7956x; 3.7956x over previous
import functools

import numpy as np
import jax
import jax.numpy as jnp
from jax import lax
from jax.experimental import pallas as pl
from jax.experimental.pallas import tpu as pltpu
from jax.experimental.pallas import tpu_sc as plsc

N = 10000
E = 320000
EMB = 128
BINS = 32
CUT = 5.0
L = 5
NCLS = 100

NCORES = 2
NSUB = 16
NWORK = NCORES * NSUB
EPW = E // NWORK
GW = 80
NWIN = EPW // GW
RPS = 624
TAIL = N - NSUB * RPS
ZB = 48

BE = 2560


_GSTEP = CUT / BINS


def _node_init_body(x_ref, emb_ref, aw_ref, ab_ref, ws_ref, wd_ref,
                    node_ref, ns_ref, nd_ref):
    xi = x_ref[...]
    iota = lax.broadcasted_iota(jnp.int32, (1, NCLS), 1)
    oh = (xi == iota).astype(jnp.float32)
    emb = jnp.dot(oh, emb_ref[...], preferred_element_type=jnp.float32)
    node = jnp.dot(emb, aw_ref[...], preferred_element_type=jnp.float32)
    node = node + ab_ref[...]
    node_ref[...] = node
    ns_ref[...] = jnp.dot(node, ws_ref[...], preferred_element_type=jnp.float32)
    nd_ref[...] = jnp.dot(node, wd_ref[...], preferred_element_type=jnp.float32)


_node_init = pl.pallas_call(
    _node_init_body,
    out_shape=(
        jax.ShapeDtypeStruct((N, EMB), jnp.float32),
        jax.ShapeDtypeStruct((N, EMB), jnp.float32),
        jax.ShapeDtypeStruct((N, EMB), jnp.float32),
    ),
)


def _edge_init_body(d_ref, ew_ref, eb_ref, out_ref):
    d = d_ref[...]
    k = lax.broadcasted_iota(jnp.int32, (1, BINS), 1).astype(jnp.float32)
    mu = (k + 1.0) * _GSTEP
    inv2v = 1.0 / (2.0 * _GSTEP)
    diff = d - mu
    ef = jnp.exp(-(diff * diff) * inv2v)
    out_ref[...] = (
        jnp.dot(ef, ew_ref[...], preferred_element_type=jnp.float32)
        + eb_ref[...]
    )


_edge_init = pl.pallas_call(
    _edge_init_body,
    grid=(E // BE,),
    in_specs=[
        pl.BlockSpec((BE, 1), lambda i: (i, 0)),
        pl.BlockSpec((BINS, EMB), lambda i: (0, 0)),
        pl.BlockSpec((1, EMB), lambda i: (0, 0)),
    ],
    out_specs=pl.BlockSpec((BE, EMB), lambda i: (i, 0)),
    out_shape=jax.ShapeDtypeStruct((E, EMB), jnp.float32),
)


def _edge_mlp_body(e_ref, gs_ref, gd_ref, we_ref, b1_ref, w2_ref, b2_ref,
                   out_ref):
    h = jnp.dot(e_ref[...], we_ref[...], preferred_element_type=jnp.float32)
    h = h + gs_ref[...] + gd_ref[...] + b1_ref[...]
    h = jnp.maximum(h, 0.0)
    out_ref[...] = (
        jnp.dot(h, w2_ref[...], preferred_element_type=jnp.float32)
        + b2_ref[...]
    )


_edge_mlp = pl.pallas_call(
    _edge_mlp_body,
    grid=(E // BE,),
    in_specs=[
        pl.BlockSpec((BE, EMB), lambda i: (i, 0)),
        pl.BlockSpec((BE, EMB), lambda i: (i, 0)),
        pl.BlockSpec((BE, EMB), lambda i: (i, 0)),
        pl.BlockSpec((EMB, EMB), lambda i: (0, 0)),
        pl.BlockSpec((1, EMB), lambda i: (0, 0)),
        pl.BlockSpec((EMB, EMB), lambda i: (0, 0)),
        pl.BlockSpec((1, EMB), lambda i: (0, 0)),
    ],
    out_specs=pl.BlockSpec((BE, EMB), lambda i: (i, 0)),
    out_shape=jax.ShapeDtypeStruct((E, EMB), jnp.float32),
)


def _node_mlp_body(node_ref, parts_ref, w1_ref, b1_ref, w2_ref, b2_ref,
                   ws_ref, wd_ref, node_o, ns_o, nd_o):
    agg = parts_ref[0] + parts_ref[1]
    h = jnp.dot(agg, w1_ref[...], preferred_element_type=jnp.float32)
    h = jnp.maximum(h + b1_ref[...], 0.0)
    node = node_ref[...] + (
        jnp.dot(h, w2_ref[...], preferred_element_type=jnp.float32)
        + b2_ref[...]
    )
    node_o[...] = node
    ns_o[...] = jnp.dot(node, ws_ref[...], preferred_element_type=jnp.float32)
    nd_o[...] = jnp.dot(node, wd_ref[...], preferred_element_type=jnp.float32)


_node_mlp = pl.pallas_call(
    _node_mlp_body,
    out_shape=(
        jax.ShapeDtypeStruct((N, EMB), jnp.float32),
        jax.ShapeDtypeStruct((N, EMB), jnp.float32),
        jax.ShapeDtypeStruct((N, EMB), jnp.float32),
    ),
)


def _readout_body(node_ref, ow1_ref, ob1_ref, ow2_ref, ob2_ref, o_ref):
    xm = jnp.mean(node_ref[...], axis=0, keepdims=True)
    v = jnp.dot(xm, ow1_ref[...], preferred_element_type=jnp.float32)
    v = v + ob1_ref[...]
    v = jnp.maximum(v, 0.0) * ow2_ref[...] + ob2_ref[...]
    o_ref[...] = jax.nn.sigmoid(v)


_readout = pl.pallas_call(
    _readout_body,
    out_shape=jax.ShapeDtypeStruct((1, 1), jnp.float32),
)


_sc_mesh = plsc.VectorSubcoreMesh(core_axis_name="c", subcore_axis_name="s")


@functools.partial(
    pl.kernel,
    out_type=(
        jax.ShapeDtypeStruct((E, EMB), jnp.float32),
        jax.ShapeDtypeStruct((E, EMB), jnp.float32),
    ),
    mesh=_sc_mesh,
    scratch_types=[
        pltpu.VMEM((NWIN, GW), jnp.int32),
        pltpu.VMEM((NWIN, GW), jnp.int32),
        pltpu.VMEM((2, GW, EMB), jnp.float32),
        pltpu.VMEM((2, GW, EMB), jnp.float32),
        pltpu.SemaphoreType.DMA((2,)),
        pltpu.SemaphoreType.DMA((2,)),
        pltpu.SemaphoreType.DMA((2,)),
        pltpu.SemaphoreType.DMA((2,)),
        pltpu.SemaphoreType.DMA,
    ],
)
def _sc_gather(ns_hbm, nd_hbm, si_hbm, di_hbm, gs_hbm, gd_hbm,
               si_v, di_v, bs_v, bd_v, gss, gsd, wss, wsd, isem):
    c = lax.axis_index("c")
    s = lax.axis_index("s")
    wid = s * NCORES + c
    base = wid * EPW

    ci = pltpu.make_async_copy(si_hbm.at[wid], si_v, isem)
    ci.start()
    cj = pltpu.make_async_copy(di_hbm.at[wid], di_v, isem)
    cj.start()
    ci.wait()
    cj.wait()

    def g_cp(tbl, idx_v, buf, j, b, sem):
        return pltpu.make_async_copy(tbl.at[idx_v.at[j]], buf.at[b], sem.at[b])

    def w_cp(out, buf, j, b, sem):
        return pltpu.make_async_copy(
            buf.at[b], out.at[pl.ds(base + j * GW, GW)], sem.at[b])

    g_cp(ns_hbm, si_v, bs_v, 0, 0, gss).start()
    g_cp(nd_hbm, di_v, bd_v, 0, 0, gsd).start()
    g_cp(ns_hbm, si_v, bs_v, 1, 1, gss).start()
    g_cp(nd_hbm, di_v, bd_v, 1, 1, gsd).start()

    def window(j, b):
        g_cp(ns_hbm, si_v, bs_v, j, b, gss).wait()
        g_cp(nd_hbm, di_v, bd_v, j, b, gsd).wait()
        w_cp(gs_hbm, bs_v, j, b, wss).start()
        w_cp(gd_hbm, bd_v, j, b, wsd).start()
        w_cp(gs_hbm, bs_v, j, b, wss).wait()
        w_cp(gd_hbm, bd_v, j, b, wsd).wait()

        @pl.when(j + 2 < NWIN)
        def _():
            g_cp(ns_hbm, si_v, bs_v, j + 2, b, gss).start()
            g_cp(nd_hbm, di_v, bd_v, j + 2, b, gsd).start()

    @pl.loop(0, NWIN // 2)
    def _(it):
        for b in range(2):
            window(it * 2 + b, b)

    if NWIN % 2:
        window(NWIN - 1, (NWIN - 1) % 2)


@functools.partial(
    pl.kernel,
    out_type=jax.ShapeDtypeStruct((NCORES, N, EMB), jnp.float32),
    mesh=_sc_mesh,
    scratch_types=[
        pltpu.VMEM_SHARED((N, EMB), jnp.float32),
        pltpu.VMEM((NWIN, GW), jnp.int32),
        pltpu.VMEM((2, GW, EMB), jnp.float32),
        pltpu.VMEM((ZB, EMB), jnp.float32),
        pltpu.SemaphoreType.DMA((2,)),
        pltpu.SemaphoreType.DMA,
    ],
)
def _sc_scatter(edge_hbm, si_hbm, out_hbm, acc, idx_v, ebuf, zbuf, lsem, msem):
    c = lax.axis_index("c")
    s = lax.axis_index("s")
    wid = s * NCORES + c
    base = wid * EPW

    zero = jnp.zeros((16,), jnp.float32)

    @pl.loop(0, ZB)
    def _(r):
        @pl.loop(0, EMB, step=16)
        def _(cc):
            zbuf[r, pl.ds(cc, 16)] = zero

    for k in range(RPS // ZB):
        pltpu.sync_copy(zbuf, acc.at[pl.ds(s * RPS + k * ZB, ZB)])

    @pl.when(s == NSUB - 1)
    def _():
        pltpu.sync_copy(zbuf.at[pl.ds(0, TAIL)], acc.at[pl.ds(NSUB * RPS, TAIL)])

    plsc.subcore_barrier()

    ci = pltpu.make_async_copy(si_hbm.at[wid], idx_v, msem)
    ci.start()
    ci.wait()

    def l_cp(j, b):
        return pltpu.make_async_copy(
            edge_hbm.at[pl.ds(base + j * GW, GW)], ebuf.at[b], lsem.at[b])

    l_cp(0, 0).start()
    l_cp(1, 1).start()

    def window(j, b):
        l_cp(j, b).wait()
        pltpu.sync_copy(ebuf.at[b], acc.at[idx_v.at[j]], add=True)

        @pl.when(j + 2 < NWIN)
        def _():
            l_cp(j + 2, b).start()

    @pl.loop(0, NWIN // 2)
    def _(it):
        for b in range(2):
            window(it * 2 + b, b)

    if NWIN % 2:
        window(NWIN - 1, (NWIN - 1) % 2)

    plsc.subcore_barrier()
    pltpu.sync_copy(acc.at[pl.ds(s * RPS, RPS)],
                    out_hbm.at[c, pl.ds(s * RPS, RPS)])

    @pl.when(s == NSUB - 1)
    def _():
        pltpu.sync_copy(acc.at[pl.ds(NSUB * RPS, TAIL)],
                        out_hbm.at[c, pl.ds(NSUB * RPS, TAIL)])


def kernel(x, edge_index, edge_attr, emb_table, atom_w, atom_b, ee_w, ee_b,
           ew1, eb1, ew2, eb2, nw1, nb1, nw2, nb2, ow1, ob1, ow2, ob2):
    x2 = x.astype(jnp.int32).reshape(N, 1)
    src = edge_index[0].astype(jnp.int32)
    dst = edge_index[1].astype(jnp.int32)
    srcw = src.reshape(NWORK, NWIN, GW)
    dstw = dst.reshape(NWORK, NWIN, GW)
    d2 = edge_attr.reshape(E, 1)

    We = [ew1[i, :EMB] for i in range(L)]
    Ws = [ew1[i, EMB:2 * EMB] for i in range(L)]
    Wd = [ew1[i, 2 * EMB:] for i in range(L)]

    node, ns, nd = _node_init(
        x2, emb_table, atom_w, atom_b.reshape(1, EMB), Ws[0], Wd[0])
    edge = _edge_init(d2, ee_w, ee_b.reshape(1, EMB))

    for i in range(L):
        gs, gd = _sc_gather(ns, nd, srcw, dstw)
        edge = _edge_mlp(edge, gs, gd, We[i], eb1[i].reshape(1, EMB),
                         ew2[i], eb2[i].reshape(1, EMB))
        parts = _sc_scatter(edge, srcw)
        j = (i + 1) % L
        node, ns, nd = _node_mlp(
            node, parts, nw1[i], nb1[i].reshape(1, EMB), nw2[i],
            nb2[i].reshape(1, EMB), Ws[j], Wd[j])

    out = _readout(node, ow1, ob1.reshape(1, 1), ow2, ob2.reshape(1, 1))
    return out.reshape(1)


# speedup vs baseline: 3.9271x; 1.0347x over previous
import functools

import numpy as np
import jax
import jax.numpy as jnp
from jax import lax
from jax.experimental import pallas as pl
from jax.experimental.pallas import tpu as pltpu
from jax.experimental.pallas import tpu_sc as plsc

N = 10000
E = 320000
EMB = 128
BINS = 32
CUT = 5.0
L = 5
NCLS = 100

NCORES = 2
NSUB = 16
NWORK = NCORES * NSUB
EPW = E // NWORK
GW = 80
NWIN = EPW // GW
RPS = 624
TAIL = N - NSUB * RPS
ZB = 16

BE = 2560


_GSTEP = CUT / BINS


def _node_init_body(x_ref, emb_ref, aw_ref, ab_ref, ws_ref, wd_ref,
                    node_ref, ns_ref, nd_ref):
    xi = x_ref[...]
    iota = lax.broadcasted_iota(jnp.int32, (1, NCLS), 1)
    oh = (xi == iota).astype(jnp.float32)
    emb = jnp.dot(oh, emb_ref[...], preferred_element_type=jnp.float32)
    node = jnp.dot(emb, aw_ref[...], preferred_element_type=jnp.float32)
    node = node + ab_ref[...]
    node_ref[...] = node
    ns_ref[...] = jnp.dot(node, ws_ref[...], preferred_element_type=jnp.float32)
    nd_ref[...] = jnp.dot(node, wd_ref[...], preferred_element_type=jnp.float32)


_node_init = pl.pallas_call(
    _node_init_body,
    out_shape=(
        jax.ShapeDtypeStruct((N, EMB), jnp.float32),
        jax.ShapeDtypeStruct((N, EMB), jnp.float32),
        jax.ShapeDtypeStruct((N, EMB), jnp.float32),
    ),
)


def _edge_init_body(d_ref, ew_ref, eb_ref, out_ref):
    d = d_ref[...]
    k = lax.broadcasted_iota(jnp.int32, (1, BINS), 1).astype(jnp.float32)
    mu = (k + 1.0) * _GSTEP
    inv2v = 1.0 / (2.0 * _GSTEP)
    diff = d - mu
    ef = jnp.exp(-(diff * diff) * inv2v)
    out_ref[...] = (
        jnp.dot(ef, ew_ref[...], preferred_element_type=jnp.float32)
        + eb_ref[...]
    )


_edge_init = pl.pallas_call(
    _edge_init_body,
    grid=(E // BE,),
    in_specs=[
        pl.BlockSpec((BE, 1), lambda i: (i, 0)),
        pl.BlockSpec((BINS, EMB), lambda i: (0, 0)),
        pl.BlockSpec((1, EMB), lambda i: (0, 0)),
    ],
    out_specs=pl.BlockSpec((BE, EMB), lambda i: (i, 0)),
    out_shape=jax.ShapeDtypeStruct((E, EMB), jnp.float32),
)


def _edge_mlp_body(e_ref, gs_ref, gd_ref, we_ref, b1_ref, w2_ref, b2_ref,
                   out_ref):
    h = jnp.dot(e_ref[...], we_ref[...], preferred_element_type=jnp.float32)
    h = h + gs_ref[...] + gd_ref[...] + b1_ref[...]
    h = jnp.maximum(h, 0.0)
    out_ref[...] = (
        jnp.dot(h, w2_ref[...], preferred_element_type=jnp.float32)
        + b2_ref[...]
    )


_edge_mlp = pl.pallas_call(
    _edge_mlp_body,
    grid=(E // BE,),
    in_specs=[
        pl.BlockSpec((BE, EMB), lambda i: (i, 0)),
        pl.BlockSpec((BE, EMB), lambda i: (i, 0)),
        pl.BlockSpec((BE, EMB), lambda i: (i, 0)),
        pl.BlockSpec((EMB, EMB), lambda i: (0, 0)),
        pl.BlockSpec((1, EMB), lambda i: (0, 0)),
        pl.BlockSpec((EMB, EMB), lambda i: (0, 0)),
        pl.BlockSpec((1, EMB), lambda i: (0, 0)),
    ],
    out_specs=pl.BlockSpec((BE, EMB), lambda i: (i, 0)),
    out_shape=jax.ShapeDtypeStruct((E, EMB), jnp.float32),
)


def _node_mlp_body(node_ref, parts_ref, w1_ref, b1_ref, w2_ref, b2_ref,
                   ws_ref, wd_ref, node_o, ns_o, nd_o):
    agg = parts_ref[0] + parts_ref[1]
    h = jnp.dot(agg, w1_ref[...], preferred_element_type=jnp.float32)
    h = jnp.maximum(h + b1_ref[...], 0.0)
    node = node_ref[...] + (
        jnp.dot(h, w2_ref[...], preferred_element_type=jnp.float32)
        + b2_ref[...]
    )
    node_o[...] = node
    ns_o[...] = jnp.dot(node, ws_ref[...], preferred_element_type=jnp.float32)
    nd_o[...] = jnp.dot(node, wd_ref[...], preferred_element_type=jnp.float32)


_node_mlp = pl.pallas_call(
    _node_mlp_body,
    out_shape=(
        jax.ShapeDtypeStruct((N, EMB), jnp.float32),
        jax.ShapeDtypeStruct((N, EMB), jnp.float32),
        jax.ShapeDtypeStruct((N, EMB), jnp.float32),
    ),
)


def _readout_body(node_ref, ow1_ref, ob1_ref, ow2_ref, ob2_ref, o_ref):
    xm = jnp.mean(node_ref[...], axis=0, keepdims=True)
    v = jnp.dot(xm, ow1_ref[...], preferred_element_type=jnp.float32)
    v = v + ob1_ref[...]
    v = jnp.maximum(v, 0.0) * ow2_ref[...] + ob2_ref[...]
    o_ref[...] = jax.nn.sigmoid(v)


_readout = pl.pallas_call(
    _readout_body,
    out_shape=jax.ShapeDtypeStruct((1, 1), jnp.float32),
)


_sc_mesh = plsc.VectorSubcoreMesh(core_axis_name="c", subcore_axis_name="s")


SLOTS = 4
SSLOTS = 3


@functools.partial(
    pl.kernel,
    out_type=(
        jax.ShapeDtypeStruct((E, EMB), jnp.float32),
        jax.ShapeDtypeStruct((E, EMB), jnp.float32),
    ),
    mesh=_sc_mesh,
    scratch_types=[
        pltpu.VMEM((NWIN, GW), jnp.int32),
        pltpu.VMEM((NWIN, GW), jnp.int32),
        pltpu.VMEM((SLOTS, GW, EMB), jnp.float32),
        pltpu.VMEM((SLOTS, GW, EMB), jnp.float32),
        pltpu.SemaphoreType.DMA((SLOTS,)),
        pltpu.SemaphoreType.DMA((SLOTS,)),
        pltpu.SemaphoreType.DMA((SLOTS,)),
        pltpu.SemaphoreType.DMA((SLOTS,)),
        pltpu.SemaphoreType.DMA,
    ],
)
def _sc_gather(ns_hbm, nd_hbm, si_hbm, di_hbm, gs_hbm, gd_hbm,
               si_v, di_v, bs_v, bd_v, gss, gsd, wss, wsd, isem):
    c = lax.axis_index("c")
    s = lax.axis_index("s")
    wid = s * NCORES + c
    base = wid * EPW

    ci = pltpu.make_async_copy(si_hbm.at[wid], si_v, isem)
    ci.start()
    cj = pltpu.make_async_copy(di_hbm.at[wid], di_v, isem)
    cj.start()
    ci.wait()
    cj.wait()

    def g_cp(tbl, idx_v, buf, j, b, sem):
        return pltpu.make_async_copy(tbl.at[idx_v.at[j]], buf.at[b], sem.at[b])

    def w_cp(out, buf, j, b, sem):
        return pltpu.make_async_copy(
            buf.at[b], out.at[pl.ds(base + j * GW, GW)], sem.at[b])

    for b in range(SLOTS):
        g_cp(ns_hbm, si_v, bs_v, b, b, gss).start()
        g_cp(nd_hbm, di_v, bd_v, b, b, gsd).start()

    def window(j, b):
        g_cp(ns_hbm, si_v, bs_v, j, b, gss).wait()
        g_cp(nd_hbm, di_v, bd_v, j, b, gsd).wait()
        w_cp(gs_hbm, bs_v, j, b, wss).start()
        w_cp(gd_hbm, bd_v, j, b, wsd).start()
        w_cp(gs_hbm, bs_v, j, b, wss).wait()
        w_cp(gd_hbm, bd_v, j, b, wsd).wait()

        @pl.when(j + SLOTS < NWIN)
        def _():
            g_cp(ns_hbm, si_v, bs_v, j + SLOTS, b, gss).start()
            g_cp(nd_hbm, di_v, bd_v, j + SLOTS, b, gsd).start()

    @pl.loop(0, NWIN // SLOTS)
    def _(it):
        for b in range(SLOTS):
            window(it * SLOTS + b, b)

    for r in range(NWIN - NWIN % SLOTS, NWIN):
        window(r, r % SLOTS)


@functools.partial(
    pl.kernel,
    out_type=jax.ShapeDtypeStruct((NCORES, N, EMB), jnp.float32),
    mesh=_sc_mesh,
    scratch_types=[
        pltpu.VMEM_SHARED((N, EMB), jnp.float32),
        pltpu.VMEM((NWIN, GW), jnp.int32),
        pltpu.VMEM((SSLOTS, GW, EMB), jnp.float32),
        pltpu.VMEM((ZB, EMB), jnp.float32),
        pltpu.SemaphoreType.DMA((SSLOTS,)),
        pltpu.SemaphoreType.DMA((SSLOTS,)),
        pltpu.SemaphoreType.DMA,
    ],
)
def _sc_scatter(edge_hbm, si_hbm, out_hbm, acc, idx_v, ebuf, zbuf, lsem,
                ssem, msem):
    c = lax.axis_index("c")
    s = lax.axis_index("s")
    wid = s * NCORES + c
    base = wid * EPW

    zero = jnp.zeros((16,), jnp.float32)

    @pl.loop(0, ZB)
    def _(r):
        @pl.loop(0, EMB, step=16)
        def _(cc):
            zbuf[r, pl.ds(cc, 16)] = zero

    for k in range(RPS // ZB):
        pltpu.sync_copy(zbuf, acc.at[pl.ds(s * RPS + k * ZB, ZB)])

    @pl.when(s == NSUB - 1)
    def _():
        pltpu.sync_copy(zbuf.at[pl.ds(0, TAIL)], acc.at[pl.ds(NSUB * RPS, TAIL)])

    plsc.subcore_barrier()

    ci = pltpu.make_async_copy(si_hbm.at[wid], idx_v, msem)
    ci.start()
    ci.wait()

    def l_cp(j, b):
        return pltpu.make_async_copy(
            edge_hbm.at[pl.ds(base + j * GW, GW)], ebuf.at[b], lsem.at[b])

    def s_cp(j, b):
        return pltpu.make_async_copy(ebuf.at[b], acc.at[idx_v.at[j]],
                                     ssem.at[b])

    for b in range(SSLOTS):
        l_cp(b, b).start()

    def window(j, b):
        l_cp(j, b).wait()
        pltpu.async_copy(ebuf.at[b], acc.at[idx_v.at[j]], ssem.at[b],
                         add=True)
        s_cp(j, b).wait()

        @pl.when(j + SSLOTS < NWIN)
        def _():
            l_cp(j + SSLOTS, b).start()

    @pl.loop(0, NWIN // SSLOTS)
    def _(it):
        for b in range(SSLOTS):
            window(it * SSLOTS + b, b)

    for r in range(NWIN - NWIN % SSLOTS, NWIN):
        window(r, r % SSLOTS)

    plsc.subcore_barrier()
    pltpu.sync_copy(acc.at[pl.ds(s * RPS, RPS)],
                    out_hbm.at[c, pl.ds(s * RPS, RPS)])

    @pl.when(s == NSUB - 1)
    def _():
        pltpu.sync_copy(acc.at[pl.ds(NSUB * RPS, TAIL)],
                        out_hbm.at[c, pl.ds(NSUB * RPS, TAIL)])


def kernel(x, edge_index, edge_attr, emb_table, atom_w, atom_b, ee_w, ee_b,
           ew1, eb1, ew2, eb2, nw1, nb1, nw2, nb2, ow1, ob1, ow2, ob2):
    x2 = x.astype(jnp.int32).reshape(N, 1)
    src = edge_index[0].astype(jnp.int32)
    dst = edge_index[1].astype(jnp.int32)
    srcw = src.reshape(NWORK, NWIN, GW)
    dstw = dst.reshape(NWORK, NWIN, GW)
    d2 = edge_attr.reshape(E, 1)

    We = [ew1[i, :EMB] for i in range(L)]
    Ws = [ew1[i, EMB:2 * EMB] for i in range(L)]
    Wd = [ew1[i, 2 * EMB:] for i in range(L)]

    node, ns, nd = _node_init(
        x2, emb_table, atom_w, atom_b.reshape(1, EMB), Ws[0], Wd[0])
    edge = _edge_init(d2, ee_w, ee_b.reshape(1, EMB))

    for i in range(L):
        gs, gd = _sc_gather(ns, nd, srcw, dstw)
        edge = _edge_mlp(edge, gs, gd, We[i], eb1[i].reshape(1, EMB),
                         ew2[i], eb2[i].reshape(1, EMB))
        parts = _sc_scatter(edge, srcw)
        j = (i + 1) % L
        node, ns, nd = _node_mlp(
            node, parts, nw1[i], nb1[i].reshape(1, EMB), nw2[i],
            nb2[i].reshape(1, EMB), Ws[j], Wd[j])

    out = _readout(node, ow1, ob1.reshape(1, 1), ow2, ob2.reshape(1, 1))
    return out.reshape(1)


# speedup vs baseline: 4.3915x; 1.1183x over previous
import functools

import numpy as np
import jax
import jax.numpy as jnp
from jax import lax
from jax.experimental import pallas as pl
from jax.experimental.pallas import tpu as pltpu
from jax.experimental.pallas import tpu_sc as plsc

N = 10000
E = 320000
EMB = 128
BINS = 32
CUT = 5.0
L = 5
NCLS = 100

NCORES = 2
NSUB = 16
NWORK = NCORES * NSUB
EPW = E // NWORK
GW = 80
NWIN = EPW // GW
RPS = 624
TAIL = N - NSUB * RPS
ZB = 16

BE = 2560


_GSTEP = CUT / BINS


def _node_init_body(x_ref, emb_ref, aw_ref, ab_ref, ws_ref, wd_ref,
                    node_ref, ns_ref, nd_ref):
    xi = x_ref[...]
    iota = lax.broadcasted_iota(jnp.int32, (1, NCLS), 1)
    oh = (xi == iota).astype(jnp.float32)
    emb = jnp.dot(oh, emb_ref[...], preferred_element_type=jnp.float32)
    node = jnp.dot(emb, aw_ref[...], preferred_element_type=jnp.float32)
    node = node + ab_ref[...]
    node_ref[...] = node
    ns_ref[...] = jnp.dot(node, ws_ref[...], preferred_element_type=jnp.float32)
    nd_ref[...] = jnp.dot(node, wd_ref[...], preferred_element_type=jnp.float32)


_node_init = pl.pallas_call(
    _node_init_body,
    out_shape=(
        jax.ShapeDtypeStruct((N, EMB), jnp.float32),
        jax.ShapeDtypeStruct((N, EMB), jnp.float32),
        jax.ShapeDtypeStruct((N, EMB), jnp.float32),
    ),
)


def _edge_init_body(d_ref, ew_ref, eb_ref, out_ref):
    d = d_ref[...]
    k = lax.broadcasted_iota(jnp.int32, (1, BINS), 1).astype(jnp.float32)
    mu = (k + 1.0) * _GSTEP
    inv2v = 1.0 / (2.0 * _GSTEP)
    diff = d - mu
    ef = jnp.exp(-(diff * diff) * inv2v)
    out_ref[...] = (
        jnp.dot(ef, ew_ref[...], preferred_element_type=jnp.float32)
        + eb_ref[...]
    )


_edge_init = pl.pallas_call(
    _edge_init_body,
    grid=(E // BE,),
    in_specs=[
        pl.BlockSpec((BE, 1), lambda i: (i, 0)),
        pl.BlockSpec((BINS, EMB), lambda i: (0, 0)),
        pl.BlockSpec((1, EMB), lambda i: (0, 0)),
    ],
    out_specs=pl.BlockSpec((BE, EMB), lambda i: (i, 0)),
    out_shape=jax.ShapeDtypeStruct((E, EMB), jnp.float32),
)


def _edge_mlp_body(e_ref, g_ref, we_ref, b1_ref, w2_ref, b2_ref,
                   out_ref):
    h = jnp.dot(e_ref[...], we_ref[...], preferred_element_type=jnp.float32)
    h = h + g_ref[...] + b1_ref[...]
    h = jnp.maximum(h, 0.0)
    out_ref[...] = (
        jnp.dot(h, w2_ref[...], preferred_element_type=jnp.float32)
        + b2_ref[...]
    )


_edge_mlp = pl.pallas_call(
    _edge_mlp_body,
    grid=(E // BE,),
    in_specs=[
        pl.BlockSpec((BE, EMB), lambda i: (i, 0)),
        pl.BlockSpec((BE, EMB), lambda i: (i, 0)),
        pl.BlockSpec((EMB, EMB), lambda i: (0, 0)),
        pl.BlockSpec((1, EMB), lambda i: (0, 0)),
        pl.BlockSpec((EMB, EMB), lambda i: (0, 0)),
        pl.BlockSpec((1, EMB), lambda i: (0, 0)),
    ],
    out_specs=pl.BlockSpec((BE, EMB), lambda i: (i, 0)),
    out_shape=jax.ShapeDtypeStruct((E, EMB), jnp.float32),
)


def _node_mlp_body(node_ref, parts_ref, w1_ref, b1_ref, w2_ref, b2_ref,
                   ws_ref, wd_ref, node_o, ns_o, nd_o):
    agg = parts_ref[0] + parts_ref[1]
    h = jnp.dot(agg, w1_ref[...], preferred_element_type=jnp.float32)
    h = jnp.maximum(h + b1_ref[...], 0.0)
    node = node_ref[...] + (
        jnp.dot(h, w2_ref[...], preferred_element_type=jnp.float32)
        + b2_ref[...]
    )
    node_o[...] = node
    ns_o[...] = jnp.dot(node, ws_ref[...], preferred_element_type=jnp.float32)
    nd_o[...] = jnp.dot(node, wd_ref[...], preferred_element_type=jnp.float32)


_node_mlp = pl.pallas_call(
    _node_mlp_body,
    out_shape=(
        jax.ShapeDtypeStruct((N, EMB), jnp.float32),
        jax.ShapeDtypeStruct((N, EMB), jnp.float32),
        jax.ShapeDtypeStruct((N, EMB), jnp.float32),
    ),
)


def _readout_body(node_ref, ow1_ref, ob1_ref, ow2_ref, ob2_ref, o_ref):
    xm = jnp.mean(node_ref[...], axis=0, keepdims=True)
    v = jnp.dot(xm, ow1_ref[...], preferred_element_type=jnp.float32)
    v = v + ob1_ref[...]
    v = jnp.maximum(v, 0.0) * ow2_ref[...] + ob2_ref[...]
    o_ref[...] = jax.nn.sigmoid(v)


_readout = pl.pallas_call(
    _readout_body,
    out_shape=jax.ShapeDtypeStruct((1, 1), jnp.float32),
)


_sc_mesh = plsc.VectorSubcoreMesh(core_axis_name="c", subcore_axis_name="s")


SLOTS = 3
SSLOTS = 3


@functools.partial(
    pl.kernel,
    out_type=jax.ShapeDtypeStruct((E, EMB), jnp.float32),
    mesh=_sc_mesh,
    scratch_types=[
        pltpu.VMEM((NWIN, GW), jnp.int32),
        pltpu.VMEM((NWIN, GW), jnp.int32),
        pltpu.VMEM_SHARED((NSUB, SLOTS, GW, EMB), jnp.float32),
        pltpu.VMEM((SLOTS, GW, EMB), jnp.float32),
        pltpu.VMEM((SLOTS, GW, EMB), jnp.float32),
        pltpu.VMEM((GW,), jnp.int32),
        pltpu.SemaphoreType.DMA((SLOTS,)),
        pltpu.SemaphoreType.DMA((SLOTS,)),
        pltpu.SemaphoreType.DMA((SLOTS,)),
        pltpu.SemaphoreType.DMA((SLOTS,)),
        pltpu.SemaphoreType.DMA((SLOTS,)),
        pltpu.SemaphoreType.DMA,
    ],
)
def _sc_gather(ns_hbm, nd_hbm, si_hbm, di_hbm, gsum_hbm,
               si_v, di_v, msh, bs_v, bd_v, idn, gss, gsd, css, ass, wss,
               isem):
    c = lax.axis_index("c")
    s = lax.axis_index("s")
    wid = s * NCORES + c
    base = wid * EPW

    for k in range(GW // 16):
        idn[pl.ds(k * 16, 16)] = lax.iota(jnp.int32, 16) + k * 16

    ci = pltpu.make_async_copy(si_hbm.at[wid], si_v, isem)
    ci.start()
    cj = pltpu.make_async_copy(di_hbm.at[wid], di_v, isem)
    cj.start()
    ci.wait()
    cj.wait()

    def gs_cp(j, b):
        return pltpu.make_async_copy(ns_hbm.at[si_v.at[j]], bs_v.at[b],
                                     gss.at[b])

    def c_cp(b):
        return pltpu.make_async_copy(bs_v.at[b], msh.at[s, b], css.at[b])

    def gd_cp(j, b):
        return pltpu.make_async_copy(nd_hbm.at[di_v.at[j]], bd_v.at[b],
                                     gsd.at[b])

    def a_cp(b):
        return pltpu.make_async_copy(bd_v.at[b], msh.at[s, b].at[idn],
                                     ass.at[b])

    def w_cp(j, b):
        return pltpu.make_async_copy(
            msh.at[s, b], gsum_hbm.at[pl.ds(base + j * GW, GW)], wss.at[b])

    for b in range(SLOTS):
        gs_cp(b, b).start()
        gd_cp(b, b).start()

    def window(j, b):
        gs_cp(j, b).wait()
        c_cp(b).start()
        gd_cp(j, b).wait()
        c_cp(b).wait()
        pltpu.async_copy(bd_v.at[b], msh.at[s, b].at[idn], ass.at[b],
                         add=True)
        a_cp(b).wait()
        w_cp(j, b).start()
        w_cp(j, b).wait()

        @pl.when(j + SLOTS < NWIN)
        def _():
            gs_cp(j + SLOTS, b).start()
            gd_cp(j + SLOTS, b).start()

    @pl.loop(0, NWIN // SLOTS)
    def _(it):
        for b in range(SLOTS):
            window(it * SLOTS + b, b)

    for r in range(NWIN - NWIN % SLOTS, NWIN):
        window(r, r % SLOTS)


@functools.partial(
    pl.kernel,
    out_type=jax.ShapeDtypeStruct((NCORES, N, EMB), jnp.float32),
    mesh=_sc_mesh,
    scratch_types=[
        pltpu.VMEM_SHARED((N, EMB), jnp.float32),
        pltpu.VMEM((NWIN, GW), jnp.int32),
        pltpu.VMEM((SSLOTS, GW, EMB), jnp.float32),
        pltpu.VMEM((ZB, EMB), jnp.float32),
        pltpu.SemaphoreType.DMA((SSLOTS,)),
        pltpu.SemaphoreType.DMA((SSLOTS,)),
        pltpu.SemaphoreType.DMA,
    ],
)
def _sc_scatter(edge_hbm, si_hbm, out_hbm, acc, idx_v, ebuf, zbuf, lsem,
                ssem, msem):
    c = lax.axis_index("c")
    s = lax.axis_index("s")
    wid = s * NCORES + c
    base = wid * EPW

    zero = jnp.zeros((16,), jnp.float32)

    @pl.loop(0, ZB)
    def _(r):
        @pl.loop(0, EMB, step=16)
        def _(cc):
            zbuf[r, pl.ds(cc, 16)] = zero

    for k in range(RPS // ZB):
        pltpu.sync_copy(zbuf, acc.at[pl.ds(s * RPS + k * ZB, ZB)])

    @pl.when(s == NSUB - 1)
    def _():
        pltpu.sync_copy(zbuf.at[pl.ds(0, TAIL)], acc.at[pl.ds(NSUB * RPS, TAIL)])

    plsc.subcore_barrier()

    ci = pltpu.make_async_copy(si_hbm.at[wid], idx_v, msem)
    ci.start()
    ci.wait()

    def l_cp(j, b):
        return pltpu.make_async_copy(
            edge_hbm.at[pl.ds(base + j * GW, GW)], ebuf.at[b], lsem.at[b])

    def s_cp(j, b):
        return pltpu.make_async_copy(ebuf.at[b], acc.at[idx_v.at[j]],
                                     ssem.at[b])

    for b in range(SSLOTS):
        l_cp(b, b).start()

    def window(j, b):
        l_cp(j, b).wait()
        pltpu.async_copy(ebuf.at[b], acc.at[idx_v.at[j]], ssem.at[b],
                         add=True)
        s_cp(j, b).wait()

        @pl.when(j + SSLOTS < NWIN)
        def _():
            l_cp(j + SSLOTS, b).start()

    @pl.loop(0, NWIN // SSLOTS)
    def _(it):
        for b in range(SSLOTS):
            window(it * SSLOTS + b, b)

    for r in range(NWIN - NWIN % SSLOTS, NWIN):
        window(r, r % SSLOTS)

    plsc.subcore_barrier()
    pltpu.sync_copy(acc.at[pl.ds(s * RPS, RPS)],
                    out_hbm.at[c, pl.ds(s * RPS, RPS)])

    @pl.when(s == NSUB - 1)
    def _():
        pltpu.sync_copy(acc.at[pl.ds(NSUB * RPS, TAIL)],
                        out_hbm.at[c, pl.ds(NSUB * RPS, TAIL)])


def kernel(x, edge_index, edge_attr, emb_table, atom_w, atom_b, ee_w, ee_b,
           ew1, eb1, ew2, eb2, nw1, nb1, nw2, nb2, ow1, ob1, ow2, ob2):
    x2 = x.astype(jnp.int32).reshape(N, 1)
    src = edge_index[0].astype(jnp.int32)
    dst = edge_index[1].astype(jnp.int32)
    srcw = src.reshape(NWORK, NWIN, GW)
    dstw = dst.reshape(NWORK, NWIN, GW)
    d2 = edge_attr.reshape(E, 1)

    We = [ew1[i, :EMB] for i in range(L)]
    Ws = [ew1[i, EMB:2 * EMB] for i in range(L)]
    Wd = [ew1[i, 2 * EMB:] for i in range(L)]

    node, ns, nd = _node_init(
        x2, emb_table, atom_w, atom_b.reshape(1, EMB), Ws[0], Wd[0])
    edge = _edge_init(d2, ee_w, ee_b.reshape(1, EMB))

    for i in range(L):
        gsum = _sc_gather(ns, nd, srcw, dstw)
        edge = _edge_mlp(edge, gsum, We[i], eb1[i].reshape(1, EMB),
                         ew2[i], eb2[i].reshape(1, EMB))
        parts = _sc_scatter(edge, srcw)
        j = (i + 1) % L
        node, ns, nd = _node_mlp(
            node, parts, nw1[i], nb1[i].reshape(1, EMB), nw2[i],
            nb2[i].reshape(1, EMB), Ws[j], Wd[j])

    out = _readout(node, ow1, ob1.reshape(1, 1), ow2, ob2.reshape(1, 1))
    return out.reshape(1)


# speedup vs baseline: 4.5073x; 1.0264x over previous
import functools

import numpy as np
import jax
import jax.numpy as jnp
from jax import lax
from jax.experimental import pallas as pl
from jax.experimental.pallas import tpu as pltpu
from jax.experimental.pallas import tpu_sc as plsc

N = 10000
E = 320000
EMB = 128
BINS = 32
CUT = 5.0
L = 5
NCLS = 100

NCORES = 2
NSUB = 16
NWORK = NCORES * NSUB
EPW = E // NWORK
GW = 80
NWIN = EPW // GW
RPS = 624
TAIL = N - NSUB * RPS
ZB = 16

BE = 2560


_GSTEP = CUT / BINS


def _node_init_body(x_ref, emb_ref, aw_ref, ab_ref, ws_ref, wd_ref,
                    node_ref, ns_ref, nd_ref):
    xi = x_ref[...]
    iota = lax.broadcasted_iota(jnp.int32, (1, NCLS), 1)
    oh = (xi == iota).astype(jnp.float32)
    emb = jnp.dot(oh, emb_ref[...], preferred_element_type=jnp.float32)
    node = jnp.dot(emb, aw_ref[...], preferred_element_type=jnp.float32)
    node = node + ab_ref[...]
    node_ref[...] = node
    ns_ref[...] = jnp.dot(node, ws_ref[...], preferred_element_type=jnp.float32)
    nd_ref[...] = jnp.dot(node, wd_ref[...], preferred_element_type=jnp.float32)


_node_init = pl.pallas_call(
    _node_init_body,
    out_shape=(
        jax.ShapeDtypeStruct((N, EMB), jnp.float32),
        jax.ShapeDtypeStruct((N, EMB), jnp.float32),
        jax.ShapeDtypeStruct((N, EMB), jnp.float32),
    ),
)


def _edge_init_body(d_ref, ew_ref, eb_ref, out_ref):
    d = d_ref[...]
    k = lax.broadcasted_iota(jnp.int32, (1, BINS), 1).astype(jnp.float32)
    mu = (k + 1.0) * _GSTEP
    inv2v = 1.0 / (2.0 * _GSTEP)
    diff = d - mu
    ef = jnp.exp(-(diff * diff) * inv2v)
    out_ref[...] = (
        jnp.dot(ef, ew_ref[...], preferred_element_type=jnp.float32)
        + eb_ref[...]
    )


_edge_init = pl.pallas_call(
    _edge_init_body,
    grid=(E // BE,),
    in_specs=[
        pl.BlockSpec((BE, 1), lambda i: (i, 0)),
        pl.BlockSpec((BINS, EMB), lambda i: (0, 0)),
        pl.BlockSpec((1, EMB), lambda i: (0, 0)),
    ],
    out_specs=pl.BlockSpec((BE, EMB), lambda i: (i, 0)),
    out_shape=jax.ShapeDtypeStruct((E, EMB), jnp.float32),
)


def _edge_mlp_body(e_ref, g_ref, we_ref, b1_ref, w2_ref, b2_ref,
                   out_ref):
    h = jnp.dot(e_ref[...], we_ref[...], preferred_element_type=jnp.float32)
    h = h + g_ref[...] + b1_ref[...]
    h = jnp.maximum(h, 0.0)
    out_ref[...] = (
        jnp.dot(h, w2_ref[...], preferred_element_type=jnp.float32)
        + b2_ref[...]
    )


_edge_mlp = pl.pallas_call(
    _edge_mlp_body,
    grid=(E // BE,),
    in_specs=[
        pl.BlockSpec((BE, EMB), lambda i: (i, 0)),
        pl.BlockSpec((BE, EMB), lambda i: (i, 0)),
        pl.BlockSpec((EMB, EMB), lambda i: (0, 0)),
        pl.BlockSpec((1, EMB), lambda i: (0, 0)),
        pl.BlockSpec((EMB, EMB), lambda i: (0, 0)),
        pl.BlockSpec((1, EMB), lambda i: (0, 0)),
    ],
    out_specs=pl.BlockSpec((BE, EMB), lambda i: (i, 0)),
    out_shape=jax.ShapeDtypeStruct((E, EMB), jnp.float32),
)


def _node_mlp_body(node_ref, parts_ref, w1_ref, b1_ref, w2_ref, b2_ref,
                   ws_ref, wd_ref, node_o, ns_o, nd_o):
    agg = parts_ref[0] + parts_ref[1]
    h = jnp.dot(agg, w1_ref[...], preferred_element_type=jnp.float32)
    h = jnp.maximum(h + b1_ref[...], 0.0)
    node = node_ref[...] + (
        jnp.dot(h, w2_ref[...], preferred_element_type=jnp.float32)
        + b2_ref[...]
    )
    node_o[...] = node
    ns_o[...] = jnp.dot(node, ws_ref[...], preferred_element_type=jnp.float32)
    nd_o[...] = jnp.dot(node, wd_ref[...], preferred_element_type=jnp.float32)


_node_mlp = pl.pallas_call(
    _node_mlp_body,
    out_shape=(
        jax.ShapeDtypeStruct((N, EMB), jnp.float32),
        jax.ShapeDtypeStruct((N, EMB), jnp.float32),
        jax.ShapeDtypeStruct((N, EMB), jnp.float32),
    ),
)


def _readout_body(node_ref, ow1_ref, ob1_ref, ow2_ref, ob2_ref, o_ref):
    xm = jnp.mean(node_ref[...], axis=0, keepdims=True)
    v = jnp.dot(xm, ow1_ref[...], preferred_element_type=jnp.float32)
    v = v + ob1_ref[...]
    v = jnp.maximum(v, 0.0) * ow2_ref[...] + ob2_ref[...]
    o_ref[...] = jax.nn.sigmoid(v)


_readout = pl.pallas_call(
    _readout_body,
    out_shape=jax.ShapeDtypeStruct((1, 1), jnp.float32),
)


_sc_mesh = plsc.VectorSubcoreMesh(core_axis_name="c", subcore_axis_name="s")


SLOTS = 3
SGW = 128
SNW = EPW // SGW
STAIL = EPW - SNW * SGW
SSLOTS = 2


@functools.partial(
    pl.kernel,
    out_type=jax.ShapeDtypeStruct((E, EMB), jnp.float32),
    mesh=_sc_mesh,
    scratch_types=[
        pltpu.VMEM((NWIN, GW), jnp.int32),
        pltpu.VMEM((NWIN, GW), jnp.int32),
        pltpu.VMEM_SHARED((NSUB, SLOTS, GW, EMB), jnp.float32),
        pltpu.VMEM((SLOTS, GW, EMB), jnp.float32),
        pltpu.VMEM((SLOTS, GW, EMB), jnp.float32),
        pltpu.VMEM((GW,), jnp.int32),
        pltpu.SemaphoreType.DMA((SLOTS,)),
        pltpu.SemaphoreType.DMA((SLOTS,)),
        pltpu.SemaphoreType.DMA((SLOTS,)),
        pltpu.SemaphoreType.DMA((SLOTS,)),
        pltpu.SemaphoreType.DMA((SLOTS,)),
        pltpu.SemaphoreType.DMA,
    ],
)
def _sc_gather(ns_hbm, nd_hbm, si_hbm, di_hbm, gsum_hbm,
               si_v, di_v, msh, bs_v, bd_v, idn, gss, gsd, css, ass, wss,
               isem):
    c = lax.axis_index("c")
    s = lax.axis_index("s")
    wid = s * NCORES + c
    base = wid * EPW

    for k in range(GW // 16):
        idn[pl.ds(k * 16, 16)] = lax.iota(jnp.int32, 16) + k * 16

    ci = pltpu.make_async_copy(si_hbm.at[wid], si_v, isem)
    ci.start()
    cj = pltpu.make_async_copy(di_hbm.at[wid], di_v, isem)
    cj.start()
    ci.wait()
    cj.wait()

    def gs_cp(j, b):
        return pltpu.make_async_copy(ns_hbm.at[si_v.at[j]], bs_v.at[b],
                                     gss.at[b])

    def gd_cp(j, b):
        return pltpu.make_async_copy(nd_hbm.at[di_v.at[j]], bd_v.at[b],
                                     gsd.at[b])

    def c_cp(b):
        return pltpu.make_async_copy(bs_v.at[b], msh.at[s, b], css.at[b])

    def a_cp(b):
        return pltpu.make_async_copy(bd_v.at[b], msh.at[s, b].at[idn],
                                     ass.at[b])

    def w_cp(j, b):
        return pltpu.make_async_copy(
            msh.at[s, b], gsum_hbm.at[pl.ds(base + j * GW, GW)], wss.at[b])

    for b in range(SLOTS):
        gs_cp(b, b).start()
        gd_cp(b, b).start()

    def window(j, b, wait_write):
        gs_cp(j, b).wait()
        if wait_write:
            w_cp(j - SLOTS, b).wait()
        c_cp(b).start()
        gd_cp(j, b).wait()
        c_cp(b).wait()
        pltpu.async_copy(bd_v.at[b], msh.at[s, b].at[idn], ass.at[b],
                         add=True)
        a_cp(b).wait()
        w_cp(j, b).start()

        @pl.when(j + SLOTS < NWIN)
        def _():
            gs_cp(j + SLOTS, b).start()
            gd_cp(j + SLOTS, b).start()

    for j in range(SLOTS):
        window(j, j, False)

    @pl.loop(1, (NWIN - SLOTS) // SLOTS + 1)
    def _(it):
        for b in range(SLOTS):
            window(it * SLOTS + b, b, True)

    for j in range(SLOTS + ((NWIN - SLOTS) // SLOTS) * SLOTS, NWIN):
        window(j, j % SLOTS, True)

    for j in range(NWIN - SLOTS, NWIN):
        w_cp(j, j % SLOTS).wait()


@functools.partial(
    pl.kernel,
    out_type=jax.ShapeDtypeStruct((NCORES, N, EMB), jnp.float32),
    mesh=_sc_mesh,
    scratch_types=[
        pltpu.VMEM_SHARED((N, EMB), jnp.float32),
        pltpu.VMEM((SSLOTS, SGW, EMB), jnp.float32),
        pltpu.VMEM((SSLOTS, SGW), jnp.int32),
        pltpu.VMEM((1, STAIL), jnp.int32),
        pltpu.VMEM((ZB, EMB), jnp.float32),
        pltpu.SemaphoreType.DMA((SSLOTS,)),
        pltpu.SemaphoreType.DMA((SSLOTS,)),
        pltpu.SemaphoreType.DMA((SSLOTS,)),
        pltpu.SemaphoreType.DMA,
    ],
)
def _sc_scatter(edge_hbm, si_hbm, st_hbm, out_hbm, acc, ebuf, iring, itail,
                zbuf, lsem, isems, ssem, msem):
    c = lax.axis_index("c")
    s = lax.axis_index("s")
    wid = s * NCORES + c
    base = wid * EPW

    zero = jnp.zeros((16,), jnp.float32)

    @pl.loop(0, ZB)
    def _(r):
        @pl.loop(0, EMB, step=16)
        def _(cc):
            zbuf[r, pl.ds(cc, 16)] = zero

    for k in range(RPS // ZB):
        pltpu.sync_copy(zbuf, acc.at[pl.ds(s * RPS + k * ZB, ZB)])

    @pl.when(s == NSUB - 1)
    def _():
        pltpu.sync_copy(zbuf.at[pl.ds(0, TAIL)], acc.at[pl.ds(NSUB * RPS, TAIL)])

    plsc.subcore_barrier()

    def i_cp(j, b):
        return pltpu.make_async_copy(si_hbm.at[wid, j], iring.at[b],
                                     isems.at[b])

    def l_cp(j, b):
        return pltpu.make_async_copy(
            edge_hbm.at[pl.ds(base + j * SGW, SGW)], ebuf.at[b], lsem.at[b])

    def s_cp(j, b):
        return pltpu.make_async_copy(ebuf.at[b], acc.at[iring.at[b]],
                                     ssem.at[b])

    for b in range(SSLOTS):
        i_cp(b, b).start()
        l_cp(b, b).start()

    def window(j, b):
        i_cp(j, b).wait()
        l_cp(j, b).wait()
        pltpu.async_copy(ebuf.at[b], acc.at[iring.at[b]], ssem.at[b],
                         add=True)
        s_cp(j, b).wait()

        @pl.when(j + SSLOTS < SNW)
        def _():
            i_cp(j + SSLOTS, b).start()
            l_cp(j + SSLOTS, b).start()

    @pl.loop(0, SNW // SSLOTS)
    def _(it):
        for b in range(SSLOTS):
            window(it * SSLOTS + b, b)

    for j in range(SNW - SNW % SSLOTS, SNW):
        window(j, j % SSLOTS)

    ct = pltpu.make_async_copy(st_hbm.at[wid], itail, msem)
    ct.start()
    ce = pltpu.make_async_copy(
        edge_hbm.at[pl.ds(base + SNW * SGW, STAIL)],
        zbuf.at[pl.ds(0, STAIL)], msem)
    ce.start()
    ct.wait()
    ce.wait()
    pltpu.sync_copy(zbuf.at[pl.ds(0, STAIL)], acc.at[itail.at[0]], add=True)

    plsc.subcore_barrier()
    pltpu.sync_copy(acc.at[pl.ds(s * RPS, RPS)],
                    out_hbm.at[c, pl.ds(s * RPS, RPS)])

    @pl.when(s == NSUB - 1)
    def _():
        pltpu.sync_copy(acc.at[pl.ds(NSUB * RPS, TAIL)],
                        out_hbm.at[c, pl.ds(NSUB * RPS, TAIL)])


def kernel(x, edge_index, edge_attr, emb_table, atom_w, atom_b, ee_w, ee_b,
           ew1, eb1, ew2, eb2, nw1, nb1, nw2, nb2, ow1, ob1, ow2, ob2):
    x2 = x.astype(jnp.int32).reshape(N, 1)
    src = edge_index[0].astype(jnp.int32)
    dst = edge_index[1].astype(jnp.int32)
    srcw = src.reshape(NWORK, NWIN, GW)
    dstw = dst.reshape(NWORK, NWIN, GW)
    src2 = src.reshape(NWORK, EPW)
    srcm = src2[:, :SNW * SGW].reshape(NWORK, SNW, SGW)
    srct = src2[:, SNW * SGW:].reshape(NWORK, 1, STAIL)
    d2 = edge_attr.reshape(E, 1)

    We = [ew1[i, :EMB] for i in range(L)]
    Ws = [ew1[i, EMB:2 * EMB] for i in range(L)]
    Wd = [ew1[i, 2 * EMB:] for i in range(L)]

    node, ns, nd = _node_init(
        x2, emb_table, atom_w, atom_b.reshape(1, EMB), Ws[0], Wd[0])
    edge = _edge_init(d2, ee_w, ee_b.reshape(1, EMB))

    for i in range(L):
        gsum = _sc_gather(ns, nd, srcw, dstw)
        edge = _edge_mlp(edge, gsum, We[i], eb1[i].reshape(1, EMB),
                         ew2[i], eb2[i].reshape(1, EMB))
        parts = _sc_scatter(edge, srcm, srct)
        j = (i + 1) % L
        node, ns, nd = _node_mlp(
            node, parts, nw1[i], nb1[i].reshape(1, EMB), nw2[i],
            nb2[i].reshape(1, EMB), Ws[j], Wd[j])

    out = _readout(node, ow1, ob1.reshape(1, 1), ow2, ob2.reshape(1, 1))
    return out.reshape(1)


# speedup vs baseline: 4.5177x; 1.0023x over previous
import functools

import numpy as np
import jax
import jax.numpy as jnp
from jax import lax
from jax.experimental import pallas as pl
from jax.experimental.pallas import tpu as pltpu
from jax.experimental.pallas import tpu_sc as plsc

N = 10000
E = 320000
EMB = 128
BINS = 32
CUT = 5.0
L = 5
NCLS = 100

NCORES = 2
NSUB = 16
NWORK = NCORES * NSUB
EPW = E // NWORK
GW = 128
NWIN = EPW // GW
GTAIL = EPW - NWIN * GW
RPS = 624
TAIL = N - NSUB * RPS
ZB = 8

BE = 2560


_GSTEP = CUT / BINS


def _node_init_body(x_ref, emb_ref, aw_ref, ab_ref, ws_ref, wd_ref,
                    node_ref, ns_ref, nd_ref):
    xi = x_ref[...]
    iota = lax.broadcasted_iota(jnp.int32, (1, NCLS), 1)
    oh = (xi == iota).astype(jnp.float32)
    emb = jnp.dot(oh, emb_ref[...], preferred_element_type=jnp.float32)
    node = jnp.dot(emb, aw_ref[...], preferred_element_type=jnp.float32)
    node = node + ab_ref[...]
    node_ref[...] = node
    ns_ref[...] = jnp.dot(node, ws_ref[...], preferred_element_type=jnp.float32)
    nd_ref[...] = jnp.dot(node, wd_ref[...], preferred_element_type=jnp.float32)


_node_init = pl.pallas_call(
    _node_init_body,
    out_shape=(
        jax.ShapeDtypeStruct((N, EMB), jnp.float32),
        jax.ShapeDtypeStruct((N, EMB), jnp.float32),
        jax.ShapeDtypeStruct((N, EMB), jnp.float32),
    ),
)


def _edge_init_body(d_ref, ew_ref, eb_ref, out_ref):
    d = d_ref[...]
    k = lax.broadcasted_iota(jnp.int32, (1, BINS), 1).astype(jnp.float32)
    mu = (k + 1.0) * _GSTEP
    inv2v = 1.0 / (2.0 * _GSTEP)
    diff = d - mu
    ef = jnp.exp(-(diff * diff) * inv2v)
    out_ref[...] = (
        jnp.dot(ef, ew_ref[...], preferred_element_type=jnp.float32)
        + eb_ref[...]
    )


_edge_init = pl.pallas_call(
    _edge_init_body,
    grid=(E // BE,),
    in_specs=[
        pl.BlockSpec((BE, 1), lambda i: (i, 0)),
        pl.BlockSpec((BINS, EMB), lambda i: (0, 0)),
        pl.BlockSpec((1, EMB), lambda i: (0, 0)),
    ],
    out_specs=pl.BlockSpec((BE, EMB), lambda i: (i, 0)),
    out_shape=jax.ShapeDtypeStruct((E, EMB), jnp.float32),
)


def _edge_mlp_body(e_ref, g_ref, we_ref, b1_ref, w2_ref, b2_ref,
                   out_ref):
    h = jnp.dot(e_ref[...], we_ref[...], preferred_element_type=jnp.float32)
    h = h + g_ref[...] + b1_ref[...]
    h = jnp.maximum(h, 0.0)
    out_ref[...] = (
        jnp.dot(h, w2_ref[...], preferred_element_type=jnp.float32)
        + b2_ref[...]
    )


_edge_mlp = pl.pallas_call(
    _edge_mlp_body,
    grid=(E // BE,),
    in_specs=[
        pl.BlockSpec((BE, EMB), lambda i: (i, 0)),
        pl.BlockSpec((BE, EMB), lambda i: (i, 0)),
        pl.BlockSpec((EMB, EMB), lambda i: (0, 0)),
        pl.BlockSpec((1, EMB), lambda i: (0, 0)),
        pl.BlockSpec((EMB, EMB), lambda i: (0, 0)),
        pl.BlockSpec((1, EMB), lambda i: (0, 0)),
    ],
    out_specs=pl.BlockSpec((BE, EMB), lambda i: (i, 0)),
    out_shape=jax.ShapeDtypeStruct((E, EMB), jnp.float32),
)


def _node_mlp_body(node_ref, parts_ref, w1_ref, b1_ref, w2_ref, b2_ref,
                   ws_ref, wd_ref, node_o, ns_o, nd_o):
    agg = parts_ref[0] + parts_ref[1]
    h = jnp.dot(agg, w1_ref[...], preferred_element_type=jnp.float32)
    h = jnp.maximum(h + b1_ref[...], 0.0)
    node = node_ref[...] + (
        jnp.dot(h, w2_ref[...], preferred_element_type=jnp.float32)
        + b2_ref[...]
    )
    node_o[...] = node
    ns_o[...] = jnp.dot(node, ws_ref[...], preferred_element_type=jnp.float32)
    nd_o[...] = jnp.dot(node, wd_ref[...], preferred_element_type=jnp.float32)


_node_mlp = pl.pallas_call(
    _node_mlp_body,
    out_shape=(
        jax.ShapeDtypeStruct((N, EMB), jnp.float32),
        jax.ShapeDtypeStruct((N, EMB), jnp.float32),
        jax.ShapeDtypeStruct((N, EMB), jnp.float32),
    ),
)


def _readout_body(node_ref, ow1_ref, ob1_ref, ow2_ref, ob2_ref, o_ref):
    xm = jnp.mean(node_ref[...], axis=0, keepdims=True)
    v = jnp.dot(xm, ow1_ref[...], preferred_element_type=jnp.float32)
    v = v + ob1_ref[...]
    v = jnp.maximum(v, 0.0) * ow2_ref[...] + ob2_ref[...]
    o_ref[...] = jax.nn.sigmoid(v)


_readout = pl.pallas_call(
    _readout_body,
    out_shape=jax.ShapeDtypeStruct((1, 1), jnp.float32),
)


_sc_mesh = plsc.VectorSubcoreMesh(core_axis_name="c", subcore_axis_name="s")


SLOTS = 2
SSLOTS = 3


@functools.partial(
    pl.kernel,
    out_type=jax.ShapeDtypeStruct((E, EMB), jnp.float32),
    mesh=_sc_mesh,
    scratch_types=[
        pltpu.VMEM((SLOTS, GW), jnp.int32),
        pltpu.VMEM((SLOTS, GW), jnp.int32),
        pltpu.VMEM((1, GTAIL), jnp.int32),
        pltpu.VMEM((1, GTAIL), jnp.int32),
        pltpu.VMEM_SHARED((NSUB, SLOTS, GW, EMB), jnp.float32),
        pltpu.VMEM((SLOTS, GW, EMB), jnp.float32),
        pltpu.VMEM((SLOTS, GW, EMB), jnp.float32),
        pltpu.VMEM((GW,), jnp.int32),
        pltpu.VMEM((1, GTAIL), jnp.int32),
        pltpu.SemaphoreType.DMA((SLOTS,)),
        pltpu.SemaphoreType.DMA((SLOTS,)),
        pltpu.SemaphoreType.DMA((SLOTS,)),
        pltpu.SemaphoreType.DMA((SLOTS,)),
        pltpu.SemaphoreType.DMA((SLOTS,)),
        pltpu.SemaphoreType.DMA((SLOTS,)),
        pltpu.SemaphoreType.DMA((SLOTS,)),
    ],
)
def _sc_gather(ns_hbm, nd_hbm, si_hbm, di_hbm, st_hbm, dt_hbm, gsum_hbm,
               si_v, di_v, sit, dit, msh, bs_v, bd_v, idn, idnt,
               gss, gsd, css, ass, wss, iss, isd):
    c = lax.axis_index("c")
    s = lax.axis_index("s")
    wid = s * NCORES + c
    base = wid * EPW

    for k in range(GW // 16):
        idn[pl.ds(k * 16, 16)] = lax.iota(jnp.int32, 16) + k * 16
    idnt[0, pl.ds(0, 16)] = lax.iota(jnp.int32, 16)

    def is_cp(j, b):
        return pltpu.make_async_copy(si_hbm.at[wid, j], si_v.at[b], iss.at[b])

    def id_cp(j, b):
        return pltpu.make_async_copy(di_hbm.at[wid, j], di_v.at[b], isd.at[b])

    def gs_cp(b):
        return pltpu.make_async_copy(ns_hbm.at[si_v.at[b]], bs_v.at[b],
                                     gss.at[b])

    def gd_cp(b):
        return pltpu.make_async_copy(nd_hbm.at[di_v.at[b]], bd_v.at[b],
                                     gsd.at[b])

    def c_cp(b):
        return pltpu.make_async_copy(bs_v.at[b], msh.at[s, b], css.at[b])

    def a_cp(b):
        return pltpu.make_async_copy(bd_v.at[b], msh.at[s, b].at[idn],
                                     ass.at[b])

    def w_cp(j, b):
        return pltpu.make_async_copy(
            msh.at[s, b], gsum_hbm.at[pl.ds(base + j * GW, GW)], wss.at[b])

    for b in range(SLOTS):
        is_cp(b, b).start()
        id_cp(b, b).start()
    for b in range(SLOTS):
        is_cp(b, b).wait()
        id_cp(b, b).wait()
        gs_cp(b).start()
        gd_cp(b).start()

    def window(j, b, wait_write):
        gs_cp(b).wait()
        gd_cp(b).wait()
        @pl.when(j + SLOTS < NWIN)
        def _():
            is_cp(j + SLOTS, b).start()
            id_cp(j + SLOTS, b).start()

        if wait_write:
            w_cp(j - SLOTS, b).wait()
        c_cp(b).start()
        c_cp(b).wait()
        pltpu.async_copy(bd_v.at[b], msh.at[s, b].at[idn], ass.at[b],
                         add=True)
        a_cp(b).wait()
        w_cp(j, b).start()

        @pl.when(j + SLOTS < NWIN)
        def _():
            is_cp(j + SLOTS, b).wait()
            id_cp(j + SLOTS, b).wait()
            gs_cp(b).start()
            gd_cp(b).start()

    for j in range(SLOTS):
        window(j, j, False)

    @pl.loop(1, (NWIN - SLOTS) // SLOTS + 1)
    def _(it):
        for b in range(SLOTS):
            window(it * SLOTS + b, b, True)

    for j in range(SLOTS + ((NWIN - SLOTS) // SLOTS) * SLOTS, NWIN):
        window(j, j % SLOTS, True)

    for j in range(NWIN - SLOTS, NWIN):
        w_cp(j, j % SLOTS).wait()

    ct = pltpu.make_async_copy(st_hbm.at[wid], sit, iss.at[0])
    ct.start()
    cu = pltpu.make_async_copy(dt_hbm.at[wid], dit, isd.at[0])
    cu.start()
    ct.wait()
    cu.wait()
    gt = pltpu.make_async_copy(ns_hbm.at[sit.at[0]],
                               bs_v.at[0].at[pl.ds(0, GTAIL)], gss.at[0])
    gt.start()
    ht = pltpu.make_async_copy(nd_hbm.at[dit.at[0]],
                               bd_v.at[0].at[pl.ds(0, GTAIL)], gsd.at[0])
    ht.start()
    gt.wait()
    ht.wait()
    pltpu.sync_copy(bs_v.at[0].at[pl.ds(0, GTAIL)],
                    msh.at[s, 0].at[pl.ds(0, GTAIL)])
    pltpu.sync_copy(bd_v.at[0].at[pl.ds(0, GTAIL)],
                    msh.at[s, 0].at[idnt.at[0]], add=True)
    pltpu.sync_copy(msh.at[s, 0].at[pl.ds(0, GTAIL)],
                    gsum_hbm.at[pl.ds(base + NWIN * GW, GTAIL)])


@functools.partial(
    pl.kernel,
    out_type=jax.ShapeDtypeStruct((NCORES, N, EMB), jnp.float32),
    mesh=_sc_mesh,
    scratch_types=[
        pltpu.VMEM_SHARED((N, EMB), jnp.float32),
        pltpu.VMEM((SSLOTS, GW, EMB), jnp.float32),
        pltpu.VMEM((SSLOTS, GW), jnp.int32),
        pltpu.VMEM((1, GTAIL), jnp.int32),
        pltpu.VMEM((ZB, EMB), jnp.float32),
        pltpu.SemaphoreType.DMA((SSLOTS,)),
        pltpu.SemaphoreType.DMA((SSLOTS,)),
        pltpu.SemaphoreType.DMA((SSLOTS,)),
        pltpu.SemaphoreType.DMA,
    ],
)
def _sc_scatter(edge_hbm, si_hbm, st_hbm, out_hbm, acc, ebuf, iring, itail,
                zbuf, lsem, isems, ssem, msem):
    c = lax.axis_index("c")
    s = lax.axis_index("s")
    wid = s * NCORES + c
    base = wid * EPW

    zero = jnp.zeros((16,), jnp.float32)

    @pl.loop(0, ZB)
    def _(r):
        @pl.loop(0, EMB, step=16)
        def _(cc):
            zbuf[r, pl.ds(cc, 16)] = zero

    for k in range(RPS // ZB):
        pltpu.sync_copy(zbuf, acc.at[pl.ds(s * RPS + k * ZB, ZB)])

    @pl.when(s == NSUB - 1)
    def _():
        for t in range(TAIL // ZB):
            pltpu.sync_copy(zbuf, acc.at[pl.ds(NSUB * RPS + t * ZB, ZB)])

    plsc.subcore_barrier()

    def i_cp(j, b):
        return pltpu.make_async_copy(si_hbm.at[wid, j], iring.at[b],
                                     isems.at[b])

    def l_cp(j, b):
        return pltpu.make_async_copy(
            edge_hbm.at[pl.ds(base + j * GW, GW)], ebuf.at[b], lsem.at[b])

    def s_cp(j, b):
        return pltpu.make_async_copy(ebuf.at[b], acc.at[iring.at[b]],
                                     ssem.at[b])

    for b in range(SSLOTS):
        i_cp(b, b).start()
        l_cp(b, b).start()

    def window(j, b):
        i_cp(j, b).wait()
        l_cp(j, b).wait()
        pltpu.async_copy(ebuf.at[b], acc.at[iring.at[b]], ssem.at[b],
                         add=True)
        s_cp(j, b).wait()

        @pl.when(j + SSLOTS < NWIN)
        def _():
            i_cp(j + SSLOTS, b).start()
            l_cp(j + SSLOTS, b).start()

    @pl.loop(0, NWIN // SSLOTS)
    def _(it):
        for b in range(SSLOTS):
            window(it * SSLOTS + b, b)

    for j in range(NWIN - NWIN % SSLOTS, NWIN):
        window(j, j % SSLOTS)

    ct = pltpu.make_async_copy(st_hbm.at[wid], itail, msem)
    ct.start()
    ce = pltpu.make_async_copy(
        edge_hbm.at[pl.ds(base + NWIN * GW, GTAIL)],
        ebuf.at[0].at[pl.ds(0, GTAIL)], msem)
    ce.start()
    ct.wait()
    ce.wait()
    pltpu.sync_copy(ebuf.at[0].at[pl.ds(0, GTAIL)], acc.at[itail.at[0]],
                    add=True)

    plsc.subcore_barrier()
    pltpu.sync_copy(acc.at[pl.ds(s * RPS, RPS)],
                    out_hbm.at[c, pl.ds(s * RPS, RPS)])

    @pl.when(s == NSUB - 1)
    def _():
        pltpu.sync_copy(acc.at[pl.ds(NSUB * RPS, TAIL)],
                        out_hbm.at[c, pl.ds(NSUB * RPS, TAIL)])


def kernel(x, edge_index, edge_attr, emb_table, atom_w, atom_b, ee_w, ee_b,
           ew1, eb1, ew2, eb2, nw1, nb1, nw2, nb2, ow1, ob1, ow2, ob2):
    x2 = x.astype(jnp.int32).reshape(N, 1)
    src = edge_index[0].astype(jnp.int32)
    dst = edge_index[1].astype(jnp.int32)
    src2 = src.reshape(NWORK, EPW)
    dst2 = dst.reshape(NWORK, EPW)
    srcm = src2[:, :NWIN * GW].reshape(NWORK, NWIN, GW)
    srct = src2[:, NWIN * GW:].reshape(NWORK, 1, GTAIL)
    dstm = dst2[:, :NWIN * GW].reshape(NWORK, NWIN, GW)
    dstt = dst2[:, NWIN * GW:].reshape(NWORK, 1, GTAIL)
    d2 = edge_attr.reshape(E, 1)

    We = [ew1[i, :EMB] for i in range(L)]
    Ws = [ew1[i, EMB:2 * EMB] for i in range(L)]
    Wd = [ew1[i, 2 * EMB:] for i in range(L)]

    node, ns, nd = _node_init(
        x2, emb_table, atom_w, atom_b.reshape(1, EMB), Ws[0], Wd[0])
    edge = _edge_init(d2, ee_w, ee_b.reshape(1, EMB))

    for i in range(L):
        gsum = _sc_gather(ns, nd, srcm, dstm, srct, dstt)
        edge = _edge_mlp(edge, gsum, We[i], eb1[i].reshape(1, EMB),
                         ew2[i], eb2[i].reshape(1, EMB))
        parts = _sc_scatter(edge, srcm, srct)
        j = (i + 1) % L
        node, ns, nd = _node_mlp(
            node, parts, nw1[i], nb1[i].reshape(1, EMB), nw2[i],
            nb2[i].reshape(1, EMB), Ws[j], Wd[j])

    out = _readout(node, ow1, ob1.reshape(1, 1), ow2, ob2.reshape(1, 1))
    return out.reshape(1)


# speedup vs baseline: 4.7711x; 1.0561x over previous
import functools

import numpy as np
import jax
import jax.numpy as jnp
from jax import lax
from jax.experimental import pallas as pl
from jax.experimental.pallas import tpu as pltpu
from jax.experimental.pallas import tpu_sc as plsc

N = 10000
E = 320000
EMB = 128
BINS = 32
CUT = 5.0
L = 5
NCLS = 100

NCORES = 2
NSUB = 16
NWORK = NCORES * NSUB
EPW = E // NWORK
GW = 128
NWIN = EPW // GW
GTAIL = EPW - NWIN * GW
RPS = 624
TAIL = N - NSUB * RPS
ZB = 8

BE = 5000


_GSTEP = CUT / BINS


def _node_init_body(x_ref, emb_ref, aw_ref, ab_ref, ws_ref, wd_ref,
                    node_ref, ns_ref, nd_ref):
    xi = x_ref[...]
    iota = lax.broadcasted_iota(jnp.int32, (1, NCLS), 1)
    oh = (xi == iota).astype(jnp.float32)
    emb = jnp.dot(oh, emb_ref[...], preferred_element_type=jnp.float32)
    node = jnp.dot(emb, aw_ref[...], preferred_element_type=jnp.float32)
    node = node + ab_ref[...]
    node_ref[...] = node
    ns_ref[...] = jnp.dot(node, ws_ref[...], preferred_element_type=jnp.float32)
    nd_ref[...] = jnp.dot(node, wd_ref[...], preferred_element_type=jnp.float32)


_node_init = pl.pallas_call(
    _node_init_body,
    out_shape=(
        jax.ShapeDtypeStruct((N, EMB), jnp.float32),
        jax.ShapeDtypeStruct((N, EMB), jnp.float32),
        jax.ShapeDtypeStruct((N, EMB), jnp.float32),
    ),
)


def _edge_init_body(d_ref, ew_ref, eb_ref, out_ref):
    d = d_ref[...]
    k = lax.broadcasted_iota(jnp.int32, (1, BINS), 1).astype(jnp.float32)
    mu = (k + 1.0) * _GSTEP
    inv2v = 1.0 / (2.0 * _GSTEP)
    diff = d - mu
    ef = jnp.exp(-(diff * diff) * inv2v)
    out_ref[...] = (
        jnp.dot(ef, ew_ref[...], preferred_element_type=jnp.float32)
        + eb_ref[...]
    )


_edge_init = pl.pallas_call(
    _edge_init_body,
    grid=(E // BE,),
    in_specs=[
        pl.BlockSpec((BE, 1), lambda i: (i, 0)),
        pl.BlockSpec((BINS, EMB), lambda i: (0, 0)),
        pl.BlockSpec((1, EMB), lambda i: (0, 0)),
    ],
    out_specs=pl.BlockSpec((BE, EMB), lambda i: (i, 0)),
    out_shape=jax.ShapeDtypeStruct((E, EMB), jnp.float32),
)


def _edge_mlp_body(e_ref, g_ref, we_ref, b1_ref, w2_ref, b2_ref,
                   out_ref):
    h = jnp.dot(e_ref[...], we_ref[...], preferred_element_type=jnp.float32)
    h = h + g_ref[...] + b1_ref[...]
    h = jnp.maximum(h, 0.0)
    out_ref[...] = (
        jnp.dot(h, w2_ref[...], preferred_element_type=jnp.float32)
        + b2_ref[...]
    )


_edge_mlp = pl.pallas_call(
    _edge_mlp_body,
    grid=(E // BE,),
    in_specs=[
        pl.BlockSpec((BE, EMB), lambda i: (i, 0)),
        pl.BlockSpec((BE, EMB), lambda i: (i, 0)),
        pl.BlockSpec((EMB, EMB), lambda i: (0, 0)),
        pl.BlockSpec((1, EMB), lambda i: (0, 0)),
        pl.BlockSpec((EMB, EMB), lambda i: (0, 0)),
        pl.BlockSpec((1, EMB), lambda i: (0, 0)),
    ],
    out_specs=pl.BlockSpec((BE, EMB), lambda i: (i, 0)),
    out_shape=jax.ShapeDtypeStruct((E, EMB), jnp.float32),
)


def _node_mlp_body(node_ref, parts_ref, w1_ref, b1_ref, w2_ref, b2_ref,
                   ws_ref, wd_ref, node_o, ns_o, nd_o):
    agg = parts_ref[0] + parts_ref[1]
    h = jnp.dot(agg, w1_ref[...], preferred_element_type=jnp.float32)
    h = jnp.maximum(h + b1_ref[...], 0.0)
    node = node_ref[...] + (
        jnp.dot(h, w2_ref[...], preferred_element_type=jnp.float32)
        + b2_ref[...]
    )
    node_o[...] = node
    ns_o[...] = jnp.dot(node, ws_ref[...], preferred_element_type=jnp.float32)
    nd_o[...] = jnp.dot(node, wd_ref[...], preferred_element_type=jnp.float32)


_node_mlp = pl.pallas_call(
    _node_mlp_body,
    out_shape=(
        jax.ShapeDtypeStruct((N, EMB), jnp.float32),
        jax.ShapeDtypeStruct((N, EMB), jnp.float32),
        jax.ShapeDtypeStruct((N, EMB), jnp.float32),
    ),
)


def _readout_body(node_ref, ow1_ref, ob1_ref, ow2_ref, ob2_ref, o_ref):
    xm = jnp.mean(node_ref[...], axis=0, keepdims=True)
    v = jnp.dot(xm, ow1_ref[...], preferred_element_type=jnp.float32)
    v = v + ob1_ref[...]
    v = jnp.maximum(v, 0.0) * ow2_ref[...] + ob2_ref[...]
    o_ref[...] = jax.nn.sigmoid(v)


_readout = pl.pallas_call(
    _readout_body,
    out_shape=jax.ShapeDtypeStruct((1, 1), jnp.float32),
)


_sc_mesh = plsc.VectorSubcoreMesh(core_axis_name="c", subcore_axis_name="s")


SLOTS = 2
SSLOTS = 3


@functools.partial(
    pl.kernel,
    out_type=jax.ShapeDtypeStruct((E, EMB), jnp.float32),
    mesh=_sc_mesh,
    scratch_types=[
        pltpu.VMEM((SLOTS, GW), jnp.int32),
        pltpu.VMEM((SLOTS, GW), jnp.int32),
        pltpu.VMEM((1, GTAIL), jnp.int32),
        pltpu.VMEM((1, GTAIL), jnp.int32),
        pltpu.VMEM_SHARED((NSUB, SLOTS, GW, EMB), jnp.float32),
        pltpu.VMEM((SLOTS, GW, EMB), jnp.float32),
        pltpu.VMEM((SLOTS, GW, EMB), jnp.float32),
        pltpu.VMEM((GW,), jnp.int32),
        pltpu.VMEM((1, GTAIL), jnp.int32),
        pltpu.SemaphoreType.DMA((SLOTS,)),
        pltpu.SemaphoreType.DMA((SLOTS,)),
        pltpu.SemaphoreType.DMA((SLOTS,)),
        pltpu.SemaphoreType.DMA((SLOTS,)),
        pltpu.SemaphoreType.DMA((SLOTS,)),
        pltpu.SemaphoreType.DMA((SLOTS,)),
        pltpu.SemaphoreType.DMA((SLOTS,)),
    ],
)
def _sc_gather(ns_hbm, nd_hbm, si_hbm, di_hbm, st_hbm, dt_hbm, gsum_hbm,
               si_v, di_v, sit, dit, msh, bs_v, bd_v, idn, idnt,
               gss, gsd, css, ass, wss, iss, isd):
    c = lax.axis_index("c")
    s = lax.axis_index("s")
    wid = s * NCORES + c
    base = wid * EPW

    for k in range(GW // 16):
        idn[pl.ds(k * 16, 16)] = lax.iota(jnp.int32, 16) + k * 16
    idnt[0, pl.ds(0, 16)] = lax.iota(jnp.int32, 16)

    def is_cp(j, b):
        return pltpu.make_async_copy(si_hbm.at[wid, j], si_v.at[b], iss.at[b])

    def id_cp(j, b):
        return pltpu.make_async_copy(di_hbm.at[wid, j], di_v.at[b], isd.at[b])

    def gs_cp(b):
        return pltpu.make_async_copy(ns_hbm.at[si_v.at[b]], bs_v.at[b],
                                     gss.at[b])

    def gd_cp(b):
        return pltpu.make_async_copy(nd_hbm.at[di_v.at[b]], bd_v.at[b],
                                     gsd.at[b])

    def c_cp(b):
        return pltpu.make_async_copy(bs_v.at[b], msh.at[s, b], css.at[b])

    def a_cp(b):
        return pltpu.make_async_copy(bd_v.at[b], msh.at[s, b].at[idn],
                                     ass.at[b])

    def w_cp(j, b):
        return pltpu.make_async_copy(
            msh.at[s, b], gsum_hbm.at[pl.ds(base + j * GW, GW)], wss.at[b])

    for b in range(SLOTS):
        is_cp(b, b).start()
        id_cp(b, b).start()
    for b in range(SLOTS):
        is_cp(b, b).wait()
        id_cp(b, b).wait()
        gs_cp(b).start()
        gd_cp(b).start()

    def window(j, b, wait_write):
        gs_cp(b).wait()
        gd_cp(b).wait()
        @pl.when(j + SLOTS < NWIN)
        def _():
            is_cp(j + SLOTS, b).start()
            id_cp(j + SLOTS, b).start()

        if wait_write:
            w_cp(j - SLOTS, b).wait()
        c_cp(b).start()
        c_cp(b).wait()
        pltpu.async_copy(bd_v.at[b], msh.at[s, b].at[idn], ass.at[b],
                         add=True)
        a_cp(b).wait()
        w_cp(j, b).start()

        @pl.when(j + SLOTS < NWIN)
        def _():
            is_cp(j + SLOTS, b).wait()
            id_cp(j + SLOTS, b).wait()
            gs_cp(b).start()
            gd_cp(b).start()

    for j in range(SLOTS):
        window(j, j, False)

    @pl.loop(1, (NWIN - SLOTS) // SLOTS + 1)
    def _(it):
        for b in range(SLOTS):
            window(it * SLOTS + b, b, True)

    for j in range(SLOTS + ((NWIN - SLOTS) // SLOTS) * SLOTS, NWIN):
        window(j, j % SLOTS, True)

    for j in range(NWIN - SLOTS, NWIN):
        w_cp(j, j % SLOTS).wait()

    ct = pltpu.make_async_copy(st_hbm.at[wid], sit, iss.at[0])
    ct.start()
    cu = pltpu.make_async_copy(dt_hbm.at[wid], dit, isd.at[0])
    cu.start()
    ct.wait()
    cu.wait()
    gt = pltpu.make_async_copy(ns_hbm.at[sit.at[0]],
                               bs_v.at[0].at[pl.ds(0, GTAIL)], gss.at[0])
    gt.start()
    ht = pltpu.make_async_copy(nd_hbm.at[dit.at[0]],
                               bd_v.at[0].at[pl.ds(0, GTAIL)], gsd.at[0])
    ht.start()
    gt.wait()
    ht.wait()
    pltpu.sync_copy(bs_v.at[0].at[pl.ds(0, GTAIL)],
                    msh.at[s, 0].at[pl.ds(0, GTAIL)])
    pltpu.sync_copy(bd_v.at[0].at[pl.ds(0, GTAIL)],
                    msh.at[s, 0].at[idnt.at[0]], add=True)
    pltpu.sync_copy(msh.at[s, 0].at[pl.ds(0, GTAIL)],
                    gsum_hbm.at[pl.ds(base + NWIN * GW, GTAIL)])


@functools.partial(
    pl.kernel,
    out_type=jax.ShapeDtypeStruct((NCORES, N, EMB), jnp.float32),
    mesh=_sc_mesh,
    scratch_types=[
        pltpu.VMEM_SHARED((N, EMB), jnp.float32),
        pltpu.VMEM((SSLOTS, GW, EMB), jnp.float32),
        pltpu.VMEM((SSLOTS, GW), jnp.int32),
        pltpu.VMEM((1, GTAIL), jnp.int32),
        pltpu.VMEM((ZB, EMB), jnp.float32),
        pltpu.SemaphoreType.DMA((SSLOTS,)),
        pltpu.SemaphoreType.DMA((SSLOTS,)),
        pltpu.SemaphoreType.DMA((SSLOTS,)),
        pltpu.SemaphoreType.DMA,
    ],
)
def _sc_scatter(edge_hbm, si_hbm, st_hbm, out_hbm, acc, ebuf, iring, itail,
                zbuf, lsem, isems, ssem, msem):
    c = lax.axis_index("c")
    s = lax.axis_index("s")
    wid = s * NCORES + c
    base = wid * EPW

    zero = jnp.zeros((16,), jnp.float32)

    @pl.loop(0, ZB)
    def _(r):
        @pl.loop(0, EMB, step=16)
        def _(cc):
            zbuf[r, pl.ds(cc, 16)] = zero

    for k in range(RPS // ZB):
        pltpu.sync_copy(zbuf, acc.at[pl.ds(s * RPS + k * ZB, ZB)])

    @pl.when(s == NSUB - 1)
    def _():
        for t in range(TAIL // ZB):
            pltpu.sync_copy(zbuf, acc.at[pl.ds(NSUB * RPS + t * ZB, ZB)])

    plsc.subcore_barrier()

    def i_cp(j, b):
        return pltpu.make_async_copy(si_hbm.at[wid, j], iring.at[b],
                                     isems.at[b])

    def l_cp(j, b):
        return pltpu.make_async_copy(
            edge_hbm.at[pl.ds(base + j * GW, GW)], ebuf.at[b], lsem.at[b])

    def s_cp(j, b):
        return pltpu.make_async_copy(ebuf.at[b], acc.at[iring.at[b]],
                                     ssem.at[b])

    for b in range(SSLOTS):
        i_cp(b, b).start()
        l_cp(b, b).start()

    def window(j, b):
        i_cp(j, b).wait()
        l_cp(j, b).wait()
        pltpu.async_copy(ebuf.at[b], acc.at[iring.at[b]], ssem.at[b],
                         add=True)
        s_cp(j, b).wait()

        @pl.when(j + SSLOTS < NWIN)
        def _():
            i_cp(j + SSLOTS, b).start()
            l_cp(j + SSLOTS, b).start()

    @pl.loop(0, NWIN // SSLOTS)
    def _(it):
        for b in range(SSLOTS):
            window(it * SSLOTS + b, b)

    for j in range(NWIN - NWIN % SSLOTS, NWIN):
        window(j, j % SSLOTS)

    ct = pltpu.make_async_copy(st_hbm.at[wid], itail, msem)
    ct.start()
    ce = pltpu.make_async_copy(
        edge_hbm.at[pl.ds(base + NWIN * GW, GTAIL)],
        ebuf.at[0].at[pl.ds(0, GTAIL)], msem)
    ce.start()
    ct.wait()
    ce.wait()
    pltpu.sync_copy(ebuf.at[0].at[pl.ds(0, GTAIL)], acc.at[itail.at[0]],
                    add=True)

    plsc.subcore_barrier()
    pltpu.sync_copy(acc.at[pl.ds(s * RPS, RPS)],
                    out_hbm.at[c, pl.ds(s * RPS, RPS)])

    @pl.when(s == NSUB - 1)
    def _():
        pltpu.sync_copy(acc.at[pl.ds(NSUB * RPS, TAIL)],
                        out_hbm.at[c, pl.ds(NSUB * RPS, TAIL)])


def kernel(x, edge_index, edge_attr, emb_table, atom_w, atom_b, ee_w, ee_b,
           ew1, eb1, ew2, eb2, nw1, nb1, nw2, nb2, ow1, ob1, ow2, ob2):
    x2 = x.astype(jnp.int32).reshape(N, 1)
    src = edge_index[0].astype(jnp.int32)
    dst = edge_index[1].astype(jnp.int32)
    src2 = src.reshape(NWORK, EPW)
    dst2 = dst.reshape(NWORK, EPW)
    srcm = src2[:, :NWIN * GW].reshape(NWORK, NWIN, GW)
    srct = src2[:, NWIN * GW:].reshape(NWORK, 1, GTAIL)
    dstm = dst2[:, :NWIN * GW].reshape(NWORK, NWIN, GW)
    dstt = dst2[:, NWIN * GW:].reshape(NWORK, 1, GTAIL)
    d2 = edge_attr.reshape(E, 1)

    We = [ew1[i, :EMB] for i in range(L)]
    Ws = [ew1[i, EMB:2 * EMB] for i in range(L)]
    Wd = [ew1[i, 2 * EMB:] for i in range(L)]

    node, ns, nd = _node_init(
        x2, emb_table, atom_w, atom_b.reshape(1, EMB), Ws[0], Wd[0])
    edge = _edge_init(d2, ee_w, ee_b.reshape(1, EMB))

    for i in range(L):
        gsum = _sc_gather(ns, nd, srcm, dstm, srct, dstt)
        edge = _edge_mlp(edge, gsum, We[i], eb1[i].reshape(1, EMB),
                         ew2[i], eb2[i].reshape(1, EMB))
        parts = _sc_scatter(edge, srcm, srct)
        j = (i + 1) % L
        node, ns, nd = _node_mlp(
            node, parts, nw1[i], nb1[i].reshape(1, EMB), nw2[i],
            nb2[i].reshape(1, EMB), Ws[j], Wd[j])

    out = _readout(node, ow1, ob1.reshape(1, 1), ow2, ob2.reshape(1, 1))
    return out.reshape(1)


# speedup vs baseline: 4.9399x; 1.0354x over previous
import functools

import numpy as np
import jax
import jax.numpy as jnp
from jax import lax
from jax.experimental import pallas as pl
from jax.experimental.pallas import tpu as pltpu
from jax.experimental.pallas import tpu_sc as plsc

N = 10000
E = 320000
EMB = 128
BINS = 32
CUT = 5.0
L = 5
NCLS = 100

NCORES = 2
NSUB = 16
NWORK = NCORES * NSUB
EPW = E // NWORK
GW = 128
NWIN = EPW // GW
GTAIL = EPW - NWIN * GW
RPS = 624
TAIL = N - NSUB * RPS
ZB = 8

BE = 8000


_GSTEP = CUT / BINS


def _node_init_body(x_ref, emb_ref, aw_ref, ab_ref, ws_ref, wd_ref,
                    node_ref, ns_ref, nd_ref):
    xi = x_ref[...]
    iota = lax.broadcasted_iota(jnp.int32, (1, NCLS), 1)
    oh = (xi == iota).astype(jnp.float32)
    emb = jnp.dot(oh, emb_ref[...], preferred_element_type=jnp.float32)
    node = jnp.dot(emb, aw_ref[...], preferred_element_type=jnp.float32)
    node = node + ab_ref[...]
    node_ref[...] = node
    ns_ref[...] = jnp.dot(node, ws_ref[...], preferred_element_type=jnp.float32)
    nd_ref[...] = jnp.dot(node, wd_ref[...], preferred_element_type=jnp.float32)


_node_init = pl.pallas_call(
    _node_init_body,
    out_shape=(
        jax.ShapeDtypeStruct((N, EMB), jnp.float32),
        jax.ShapeDtypeStruct((N, EMB), jnp.float32),
        jax.ShapeDtypeStruct((N, EMB), jnp.float32),
    ),
)


def _edge_init_body(d_ref, ew_ref, eb_ref, out_ref):
    d = d_ref[...]
    k = lax.broadcasted_iota(jnp.int32, (1, BINS), 1).astype(jnp.float32)
    mu = (k + 1.0) * _GSTEP
    inv2v = 1.0 / (2.0 * _GSTEP)
    diff = d - mu
    ef = jnp.exp(-(diff * diff) * inv2v)
    out_ref[...] = (
        jnp.dot(ef, ew_ref[...], preferred_element_type=jnp.float32)
        + eb_ref[...]
    )


_edge_init = pl.pallas_call(
    _edge_init_body,
    grid=(E // BE,),
    in_specs=[
        pl.BlockSpec((BE, 1), lambda i: (i, 0)),
        pl.BlockSpec((BINS, EMB), lambda i: (0, 0)),
        pl.BlockSpec((1, EMB), lambda i: (0, 0)),
    ],
    out_specs=pl.BlockSpec((BE, EMB), lambda i: (i, 0)),
    out_shape=jax.ShapeDtypeStruct((E, EMB), jnp.float32),
)


def _edge_mlp_body(e_ref, g_ref, we_ref, b1_ref, w2_ref, b2_ref,
                   out_ref):
    h = jnp.dot(e_ref[...], we_ref[...], preferred_element_type=jnp.float32)
    h = h + g_ref[...] + b1_ref[...]
    h = jnp.maximum(h, 0.0)
    out_ref[...] = (
        jnp.dot(h, w2_ref[...], preferred_element_type=jnp.float32)
        + b2_ref[...]
    )


_edge_mlp = pl.pallas_call(
    _edge_mlp_body,
    grid=(E // BE,),
    in_specs=[
        pl.BlockSpec((BE, EMB), lambda i: (i, 0)),
        pl.BlockSpec((BE, EMB), lambda i: (i, 0)),
        pl.BlockSpec((EMB, EMB), lambda i: (0, 0)),
        pl.BlockSpec((1, EMB), lambda i: (0, 0)),
        pl.BlockSpec((EMB, EMB), lambda i: (0, 0)),
        pl.BlockSpec((1, EMB), lambda i: (0, 0)),
    ],
    out_specs=pl.BlockSpec((BE, EMB), lambda i: (i, 0)),
    out_shape=jax.ShapeDtypeStruct((E, EMB), jnp.float32),
)


def _node_mlp_body(node_ref, parts_ref, w1_ref, b1_ref, w2_ref, b2_ref,
                   ws_ref, wd_ref, node_o, ns_o, nd_o):
    agg = parts_ref[0] + parts_ref[1]
    h = jnp.dot(agg, w1_ref[...], preferred_element_type=jnp.float32)
    h = jnp.maximum(h + b1_ref[...], 0.0)
    node = node_ref[...] + (
        jnp.dot(h, w2_ref[...], preferred_element_type=jnp.float32)
        + b2_ref[...]
    )
    node_o[...] = node
    ns_o[...] = jnp.dot(node, ws_ref[...], preferred_element_type=jnp.float32)
    nd_o[...] = jnp.dot(node, wd_ref[...], preferred_element_type=jnp.float32)


_node_mlp = pl.pallas_call(
    _node_mlp_body,
    out_shape=(
        jax.ShapeDtypeStruct((N, EMB), jnp.float32),
        jax.ShapeDtypeStruct((N, EMB), jnp.float32),
        jax.ShapeDtypeStruct((N, EMB), jnp.float32),
    ),
)


def _readout_body(node_ref, ow1_ref, ob1_ref, ow2_ref, ob2_ref, o_ref):
    xm = jnp.mean(node_ref[...], axis=0, keepdims=True)
    v = jnp.dot(xm, ow1_ref[...], preferred_element_type=jnp.float32)
    v = v + ob1_ref[...]
    v = jnp.maximum(v, 0.0) * ow2_ref[...] + ob2_ref[...]
    o_ref[...] = jax.nn.sigmoid(v)


_readout = pl.pallas_call(
    _readout_body,
    out_shape=jax.ShapeDtypeStruct((1, 1), jnp.float32),
)


_sc_mesh = plsc.VectorSubcoreMesh(core_axis_name="c", subcore_axis_name="s")


SLOTS = 2
SSLOTS = 3


@functools.partial(
    pl.kernel,
    out_type=jax.ShapeDtypeStruct((E, EMB), jnp.float32),
    mesh=_sc_mesh,
    scratch_types=[
        pltpu.VMEM((SLOTS, GW), jnp.int32),
        pltpu.VMEM((SLOTS, GW), jnp.int32),
        pltpu.VMEM((1, GTAIL), jnp.int32),
        pltpu.VMEM((1, GTAIL), jnp.int32),
        pltpu.VMEM_SHARED((NSUB, SLOTS, GW, EMB), jnp.float32),
        pltpu.VMEM((SLOTS, GW, EMB), jnp.float32),
        pltpu.VMEM((SLOTS, GW, EMB), jnp.float32),
        pltpu.VMEM((GW,), jnp.int32),
        pltpu.VMEM((1, GTAIL), jnp.int32),
        pltpu.SemaphoreType.DMA((SLOTS,)),
        pltpu.SemaphoreType.DMA((SLOTS,)),
        pltpu.SemaphoreType.DMA((SLOTS,)),
        pltpu.SemaphoreType.DMA((SLOTS,)),
        pltpu.SemaphoreType.DMA((SLOTS,)),
        pltpu.SemaphoreType.DMA((SLOTS,)),
        pltpu.SemaphoreType.DMA((SLOTS,)),
    ],
)
def _sc_gather(ns_hbm, nd_hbm, si_hbm, di_hbm, st_hbm, dt_hbm, gsum_hbm,
               si_v, di_v, sit, dit, msh, bs_v, bd_v, idn, idnt,
               gss, gsd, css, ass, wss, iss, isd):
    c = lax.axis_index("c")
    s = lax.axis_index("s")
    wid = s * NCORES + c
    base = wid * EPW

    for k in range(GW // 16):
        idn[pl.ds(k * 16, 16)] = lax.iota(jnp.int32, 16) + k * 16
    idnt[0, pl.ds(0, 16)] = lax.iota(jnp.int32, 16)

    def is_cp(j, b):
        return pltpu.make_async_copy(si_hbm.at[wid, j], si_v.at[b], iss.at[b])

    def id_cp(j, b):
        return pltpu.make_async_copy(di_hbm.at[wid, j], di_v.at[b], isd.at[b])

    def gs_cp(b):
        return pltpu.make_async_copy(ns_hbm.at[si_v.at[b]], bs_v.at[b],
                                     gss.at[b])

    def gd_cp(b):
        return pltpu.make_async_copy(nd_hbm.at[di_v.at[b]], bd_v.at[b],
                                     gsd.at[b])

    def c_cp(b):
        return pltpu.make_async_copy(bs_v.at[b], msh.at[s, b], css.at[b])

    def a_cp(b):
        return pltpu.make_async_copy(bd_v.at[b], msh.at[s, b].at[idn],
                                     ass.at[b])

    def w_cp(j, b):
        return pltpu.make_async_copy(
            msh.at[s, b], gsum_hbm.at[pl.ds(base + j * GW, GW)], wss.at[b])

    for b in range(SLOTS):
        is_cp(b, b).start()
        id_cp(b, b).start()
    for b in range(SLOTS):
        is_cp(b, b).wait()
        id_cp(b, b).wait()
        gs_cp(b).start()
        gd_cp(b).start()

    def window(j, b, wait_write):
        gs_cp(b).wait()
        gd_cp(b).wait()
        @pl.when(j + SLOTS < NWIN)
        def _():
            is_cp(j + SLOTS, b).start()
            id_cp(j + SLOTS, b).start()

        if wait_write:
            w_cp(j - SLOTS, b).wait()
        c_cp(b).start()
        c_cp(b).wait()
        pltpu.async_copy(bd_v.at[b], msh.at[s, b].at[idn], ass.at[b],
                         add=True)
        a_cp(b).wait()
        w_cp(j, b).start()

        @pl.when(j + SLOTS < NWIN)
        def _():
            is_cp(j + SLOTS, b).wait()
            id_cp(j + SLOTS, b).wait()
            gs_cp(b).start()
            gd_cp(b).start()

    for j in range(SLOTS):
        window(j, j, False)

    @pl.loop(1, (NWIN - SLOTS) // SLOTS + 1)
    def _(it):
        for b in range(SLOTS):
            window(it * SLOTS + b, b, True)

    for j in range(SLOTS + ((NWIN - SLOTS) // SLOTS) * SLOTS, NWIN):
        window(j, j % SLOTS, True)

    for j in range(NWIN - SLOTS, NWIN):
        w_cp(j, j % SLOTS).wait()

    ct = pltpu.make_async_copy(st_hbm.at[wid], sit, iss.at[0])
    ct.start()
    cu = pltpu.make_async_copy(dt_hbm.at[wid], dit, isd.at[0])
    cu.start()
    ct.wait()
    cu.wait()
    gt = pltpu.make_async_copy(ns_hbm.at[sit.at[0]],
                               bs_v.at[0].at[pl.ds(0, GTAIL)], gss.at[0])
    gt.start()
    ht = pltpu.make_async_copy(nd_hbm.at[dit.at[0]],
                               bd_v.at[0].at[pl.ds(0, GTAIL)], gsd.at[0])
    ht.start()
    gt.wait()
    ht.wait()
    pltpu.sync_copy(bs_v.at[0].at[pl.ds(0, GTAIL)],
                    msh.at[s, 0].at[pl.ds(0, GTAIL)])
    pltpu.sync_copy(bd_v.at[0].at[pl.ds(0, GTAIL)],
                    msh.at[s, 0].at[idnt.at[0]], add=True)
    pltpu.sync_copy(msh.at[s, 0].at[pl.ds(0, GTAIL)],
                    gsum_hbm.at[pl.ds(base + NWIN * GW, GTAIL)])


@functools.partial(
    pl.kernel,
    out_type=jax.ShapeDtypeStruct((NCORES, N, EMB), jnp.float32),
    mesh=_sc_mesh,
    scratch_types=[
        pltpu.VMEM_SHARED((N, EMB), jnp.float32),
        pltpu.VMEM((SSLOTS, GW, EMB), jnp.float32),
        pltpu.VMEM((SSLOTS, GW), jnp.int32),
        pltpu.VMEM((1, GTAIL), jnp.int32),
        pltpu.VMEM((ZB, EMB), jnp.float32),
        pltpu.SemaphoreType.DMA((SSLOTS,)),
        pltpu.SemaphoreType.DMA((SSLOTS,)),
        pltpu.SemaphoreType.DMA((SSLOTS,)),
        pltpu.SemaphoreType.DMA,
    ],
)
def _sc_scatter(edge_hbm, si_hbm, st_hbm, out_hbm, acc, ebuf, iring, itail,
                zbuf, lsem, isems, ssem, msem):
    c = lax.axis_index("c")
    s = lax.axis_index("s")
    wid = s * NCORES + c
    base = wid * EPW

    zero = jnp.zeros((16,), jnp.float32)

    @pl.loop(0, ZB)
    def _(r):
        @pl.loop(0, EMB, step=16)
        def _(cc):
            zbuf[r, pl.ds(cc, 16)] = zero

    for k in range(RPS // ZB):
        pltpu.sync_copy(zbuf, acc.at[pl.ds(s * RPS + k * ZB, ZB)])

    @pl.when(s == NSUB - 1)
    def _():
        for t in range(TAIL // ZB):
            pltpu.sync_copy(zbuf, acc.at[pl.ds(NSUB * RPS + t * ZB, ZB)])

    plsc.subcore_barrier()

    def i_cp(j, b):
        return pltpu.make_async_copy(si_hbm.at[wid, j], iring.at[b],
                                     isems.at[b])

    def l_cp(j, b):
        return pltpu.make_async_copy(
            edge_hbm.at[pl.ds(base + j * GW, GW)], ebuf.at[b], lsem.at[b])

    def s_cp(j, b):
        return pltpu.make_async_copy(ebuf.at[b], acc.at[iring.at[b]],
                                     ssem.at[b])

    for b in range(SSLOTS):
        i_cp(b, b).start()
        l_cp(b, b).start()

    def window(j, b):
        i_cp(j, b).wait()
        l_cp(j, b).wait()
        pltpu.async_copy(ebuf.at[b], acc.at[iring.at[b]], ssem.at[b],
                         add=True)
        s_cp(j, b).wait()

        @pl.when(j + SSLOTS < NWIN)
        def _():
            i_cp(j + SSLOTS, b).start()
            l_cp(j + SSLOTS, b).start()

    @pl.loop(0, NWIN // SSLOTS)
    def _(it):
        for b in range(SSLOTS):
            window(it * SSLOTS + b, b)

    for j in range(NWIN - NWIN % SSLOTS, NWIN):
        window(j, j % SSLOTS)

    ct = pltpu.make_async_copy(st_hbm.at[wid], itail, msem)
    ct.start()
    ce = pltpu.make_async_copy(
        edge_hbm.at[pl.ds(base + NWIN * GW, GTAIL)],
        ebuf.at[0].at[pl.ds(0, GTAIL)], msem)
    ce.start()
    ct.wait()
    ce.wait()
    pltpu.sync_copy(ebuf.at[0].at[pl.ds(0, GTAIL)], acc.at[itail.at[0]],
                    add=True)

    plsc.subcore_barrier()
    pltpu.sync_copy(acc.at[pl.ds(s * RPS, RPS)],
                    out_hbm.at[c, pl.ds(s * RPS, RPS)])

    @pl.when(s == NSUB - 1)
    def _():
        pltpu.sync_copy(acc.at[pl.ds(NSUB * RPS, TAIL)],
                        out_hbm.at[c, pl.ds(NSUB * RPS, TAIL)])


def kernel(x, edge_index, edge_attr, emb_table, atom_w, atom_b, ee_w, ee_b,
           ew1, eb1, ew2, eb2, nw1, nb1, nw2, nb2, ow1, ob1, ow2, ob2):
    x2 = x.astype(jnp.int32).reshape(N, 1)
    src = edge_index[0].astype(jnp.int32)
    dst = edge_index[1].astype(jnp.int32)
    src2 = src.reshape(NWORK, EPW)
    dst2 = dst.reshape(NWORK, EPW)
    srcm = src2[:, :NWIN * GW].reshape(NWORK, NWIN, GW)
    srct = src2[:, NWIN * GW:].reshape(NWORK, 1, GTAIL)
    dstm = dst2[:, :NWIN * GW].reshape(NWORK, NWIN, GW)
    dstt = dst2[:, NWIN * GW:].reshape(NWORK, 1, GTAIL)
    d2 = edge_attr.reshape(E, 1)

    We = [ew1[i, :EMB] for i in range(L)]
    Ws = [ew1[i, EMB:2 * EMB] for i in range(L)]
    Wd = [ew1[i, 2 * EMB:] for i in range(L)]

    node, ns, nd = _node_init(
        x2, emb_table, atom_w, atom_b.reshape(1, EMB), Ws[0], Wd[0])
    edge = _edge_init(d2, ee_w, ee_b.reshape(1, EMB))

    for i in range(L):
        gsum = _sc_gather(ns, nd, srcm, dstm, srct, dstt)
        edge = _edge_mlp(edge, gsum, We[i], eb1[i].reshape(1, EMB),
                         ew2[i], eb2[i].reshape(1, EMB))
        parts = _sc_scatter(edge, srcm, srct)
        j = (i + 1) % L
        node, ns, nd = _node_mlp(
            node, parts, nw1[i], nb1[i].reshape(1, EMB), nw2[i],
            nb2[i].reshape(1, EMB), Ws[j], Wd[j])

    out = _readout(node, ow1, ob1.reshape(1, 1), ow2, ob2.reshape(1, 1))
    return out.reshape(1)


# speedup vs baseline: 4.9501x; 1.0021x over previous
import functools

import numpy as np
import jax
import jax.numpy as jnp
from jax import lax
from jax.experimental import pallas as pl
from jax.experimental.pallas import tpu as pltpu
from jax.experimental.pallas import tpu_sc as plsc

N = 10000
E = 320000
EMB = 128
BINS = 32
CUT = 5.0
L = 5
NCLS = 100

NCORES = 2
NSUB = 16
NWORK = NCORES * NSUB
EPW = E // NWORK
GW = 128
NWIN = EPW // GW
GTAIL = EPW - NWIN * GW
RPS = 624
TAIL = N - NSUB * RPS
ZB = 8

BE = 10000


_GSTEP = CUT / BINS


def _node_init_body(x_ref, emb_ref, aw_ref, ab_ref, ws_ref, wd_ref,
                    node_ref, ns_ref, nd_ref):
    xi = x_ref[...]
    iota = lax.broadcasted_iota(jnp.int32, (1, NCLS), 1)
    oh = (xi == iota).astype(jnp.float32)
    emb = jnp.dot(oh, emb_ref[...], preferred_element_type=jnp.float32)
    node = jnp.dot(emb, aw_ref[...], preferred_element_type=jnp.float32)
    node = node + ab_ref[...]
    node_ref[...] = node
    ns_ref[...] = jnp.dot(node, ws_ref[...], preferred_element_type=jnp.float32)
    nd_ref[...] = jnp.dot(node, wd_ref[...], preferred_element_type=jnp.float32)


_node_init = pl.pallas_call(
    _node_init_body,
    out_shape=(
        jax.ShapeDtypeStruct((N, EMB), jnp.float32),
        jax.ShapeDtypeStruct((N, EMB), jnp.float32),
        jax.ShapeDtypeStruct((N, EMB), jnp.float32),
    ),
)


def _edge_init_body(d_ref, ew_ref, eb_ref, out_ref):
    d = d_ref[...]
    k = lax.broadcasted_iota(jnp.int32, (1, BINS), 1).astype(jnp.float32)
    mu = (k + 1.0) * _GSTEP
    inv2v = 1.0 / (2.0 * _GSTEP)
    diff = d - mu
    ef = jnp.exp(-(diff * diff) * inv2v)
    out_ref[...] = (
        jnp.dot(ef, ew_ref[...], preferred_element_type=jnp.float32)
        + eb_ref[...]
    )


_edge_init = pl.pallas_call(
    _edge_init_body,
    grid=(E // BE,),
    in_specs=[
        pl.BlockSpec((BE, 1), lambda i: (i, 0)),
        pl.BlockSpec((BINS, EMB), lambda i: (0, 0)),
        pl.BlockSpec((1, EMB), lambda i: (0, 0)),
    ],
    out_specs=pl.BlockSpec((BE, EMB), lambda i: (i, 0)),
    out_shape=jax.ShapeDtypeStruct((E, EMB), jnp.float32),
)


def _edge_mlp_body(e_ref, g_ref, we_ref, b1_ref, w2_ref, b2_ref,
                   out_ref):
    h = jnp.dot(e_ref[...], we_ref[...], preferred_element_type=jnp.float32)
    h = h + g_ref[...] + b1_ref[...]
    h = jnp.maximum(h, 0.0)
    out_ref[...] = (
        jnp.dot(h, w2_ref[...], preferred_element_type=jnp.float32)
        + b2_ref[...]
    )


_edge_mlp = pl.pallas_call(
    _edge_mlp_body,
    grid=(E // BE,),
    in_specs=[
        pl.BlockSpec((BE, EMB), lambda i: (i, 0)),
        pl.BlockSpec((BE, EMB), lambda i: (i, 0)),
        pl.BlockSpec((EMB, EMB), lambda i: (0, 0)),
        pl.BlockSpec((1, EMB), lambda i: (0, 0)),
        pl.BlockSpec((EMB, EMB), lambda i: (0, 0)),
        pl.BlockSpec((1, EMB), lambda i: (0, 0)),
    ],
    out_specs=pl.BlockSpec((BE, EMB), lambda i: (i, 0)),
    out_shape=jax.ShapeDtypeStruct((E, EMB), jnp.float32),
)


def _node_mlp_body(node_ref, parts_ref, w1_ref, b1_ref, w2_ref, b2_ref,
                   ws_ref, wd_ref, node_o, ns_o, nd_o):
    agg = parts_ref[0] + parts_ref[1]
    h = jnp.dot(agg, w1_ref[...], preferred_element_type=jnp.float32)
    h = jnp.maximum(h + b1_ref[...], 0.0)
    node = node_ref[...] + (
        jnp.dot(h, w2_ref[...], preferred_element_type=jnp.float32)
        + b2_ref[...]
    )
    node_o[...] = node
    ns_o[...] = jnp.dot(node, ws_ref[...], preferred_element_type=jnp.float32)
    nd_o[...] = jnp.dot(node, wd_ref[...], preferred_element_type=jnp.float32)


_node_mlp = pl.pallas_call(
    _node_mlp_body,
    out_shape=(
        jax.ShapeDtypeStruct((N, EMB), jnp.float32),
        jax.ShapeDtypeStruct((N, EMB), jnp.float32),
        jax.ShapeDtypeStruct((N, EMB), jnp.float32),
    ),
)


def _readout_body(node_ref, ow1_ref, ob1_ref, ow2_ref, ob2_ref, o_ref):
    xm = jnp.mean(node_ref[...], axis=0, keepdims=True)
    v = jnp.dot(xm, ow1_ref[...], preferred_element_type=jnp.float32)
    v = v + ob1_ref[...]
    v = jnp.maximum(v, 0.0) * ow2_ref[...] + ob2_ref[...]
    o_ref[...] = jax.nn.sigmoid(v)


_readout = pl.pallas_call(
    _readout_body,
    out_shape=jax.ShapeDtypeStruct((1, 1), jnp.float32),
)


_sc_mesh = plsc.VectorSubcoreMesh(core_axis_name="c", subcore_axis_name="s")


SLOTS = 2
SSLOTS = 3


@functools.partial(
    pl.kernel,
    out_type=jax.ShapeDtypeStruct((E, EMB), jnp.float32),
    mesh=_sc_mesh,
    scratch_types=[
        pltpu.VMEM((SLOTS, GW), jnp.int32),
        pltpu.VMEM((SLOTS, GW), jnp.int32),
        pltpu.VMEM((1, GTAIL), jnp.int32),
        pltpu.VMEM((1, GTAIL), jnp.int32),
        pltpu.VMEM_SHARED((NSUB, SLOTS, GW, EMB), jnp.float32),
        pltpu.VMEM((SLOTS, GW, EMB), jnp.float32),
        pltpu.VMEM((SLOTS, GW, EMB), jnp.float32),
        pltpu.VMEM((GW,), jnp.int32),
        pltpu.VMEM((1, GTAIL), jnp.int32),
        pltpu.SemaphoreType.DMA((SLOTS,)),
        pltpu.SemaphoreType.DMA((SLOTS,)),
        pltpu.SemaphoreType.DMA((SLOTS,)),
        pltpu.SemaphoreType.DMA((SLOTS,)),
        pltpu.SemaphoreType.DMA((SLOTS,)),
        pltpu.SemaphoreType.DMA((SLOTS,)),
        pltpu.SemaphoreType.DMA((SLOTS,)),
    ],
)
def _sc_gather(ns_hbm, nd_hbm, si_hbm, di_hbm, st_hbm, dt_hbm, gsum_hbm,
               si_v, di_v, sit, dit, msh, bs_v, bd_v, idn, idnt,
               gss, gsd, css, ass, wss, iss, isd):
    c = lax.axis_index("c")
    s = lax.axis_index("s")
    wid = s * NCORES + c
    base = wid * EPW

    for k in range(GW // 16):
        idn[pl.ds(k * 16, 16)] = lax.iota(jnp.int32, 16) + k * 16
    idnt[0, pl.ds(0, 16)] = lax.iota(jnp.int32, 16)

    def is_cp(j, b):
        return pltpu.make_async_copy(si_hbm.at[wid, j], si_v.at[b], iss.at[b])

    def id_cp(j, b):
        return pltpu.make_async_copy(di_hbm.at[wid, j], di_v.at[b], isd.at[b])

    def gs_cp(b):
        return pltpu.make_async_copy(ns_hbm.at[si_v.at[b]], bs_v.at[b],
                                     gss.at[b])

    def gd_cp(b):
        return pltpu.make_async_copy(nd_hbm.at[di_v.at[b]], bd_v.at[b],
                                     gsd.at[b])

    def c_cp(b):
        return pltpu.make_async_copy(bs_v.at[b], msh.at[s, b], css.at[b])

    def a_cp(b):
        return pltpu.make_async_copy(bd_v.at[b], msh.at[s, b].at[idn],
                                     ass.at[b])

    def w_cp(j, b):
        return pltpu.make_async_copy(
            msh.at[s, b], gsum_hbm.at[pl.ds(base + j * GW, GW)], wss.at[b])

    for b in range(SLOTS):
        is_cp(b, b).start()
        id_cp(b, b).start()
    for b in range(SLOTS):
        is_cp(b, b).wait()
        id_cp(b, b).wait()
        gs_cp(b).start()
        gd_cp(b).start()

    def window(j, b, wait_write):
        gs_cp(b).wait()
        gd_cp(b).wait()
        @pl.when(j + SLOTS < NWIN)
        def _():
            is_cp(j + SLOTS, b).start()
            id_cp(j + SLOTS, b).start()

        if wait_write:
            w_cp(j - SLOTS, b).wait()
        c_cp(b).start()
        c_cp(b).wait()
        pltpu.async_copy(bd_v.at[b], msh.at[s, b].at[idn], ass.at[b],
                         add=True)
        a_cp(b).wait()
        w_cp(j, b).start()

        @pl.when(j + SLOTS < NWIN)
        def _():
            is_cp(j + SLOTS, b).wait()
            id_cp(j + SLOTS, b).wait()
            gs_cp(b).start()
            gd_cp(b).start()

    for j in range(SLOTS):
        window(j, j, False)

    @pl.loop(1, (NWIN - SLOTS) // SLOTS + 1)
    def _(it):
        for b in range(SLOTS):
            window(it * SLOTS + b, b, True)

    for j in range(SLOTS + ((NWIN - SLOTS) // SLOTS) * SLOTS, NWIN):
        window(j, j % SLOTS, True)

    for j in range(NWIN - SLOTS, NWIN):
        w_cp(j, j % SLOTS).wait()

    ct = pltpu.make_async_copy(st_hbm.at[wid], sit, iss.at[0])
    ct.start()
    cu = pltpu.make_async_copy(dt_hbm.at[wid], dit, isd.at[0])
    cu.start()
    ct.wait()
    cu.wait()
    gt = pltpu.make_async_copy(ns_hbm.at[sit.at[0]],
                               bs_v.at[0].at[pl.ds(0, GTAIL)], gss.at[0])
    gt.start()
    ht = pltpu.make_async_copy(nd_hbm.at[dit.at[0]],
                               bd_v.at[0].at[pl.ds(0, GTAIL)], gsd.at[0])
    ht.start()
    gt.wait()
    ht.wait()
    pltpu.sync_copy(bs_v.at[0].at[pl.ds(0, GTAIL)],
                    msh.at[s, 0].at[pl.ds(0, GTAIL)])
    pltpu.sync_copy(bd_v.at[0].at[pl.ds(0, GTAIL)],
                    msh.at[s, 0].at[idnt.at[0]], add=True)
    pltpu.sync_copy(msh.at[s, 0].at[pl.ds(0, GTAIL)],
                    gsum_hbm.at[pl.ds(base + NWIN * GW, GTAIL)])


@functools.partial(
    pl.kernel,
    out_type=jax.ShapeDtypeStruct((NCORES, N, EMB), jnp.float32),
    mesh=_sc_mesh,
    scratch_types=[
        pltpu.VMEM_SHARED((N, EMB), jnp.float32),
        pltpu.VMEM((SSLOTS, GW, EMB), jnp.float32),
        pltpu.VMEM((SSLOTS, GW), jnp.int32),
        pltpu.VMEM((1, GTAIL), jnp.int32),
        pltpu.VMEM((ZB, EMB), jnp.float32),
        pltpu.SemaphoreType.DMA((SSLOTS,)),
        pltpu.SemaphoreType.DMA((SSLOTS,)),
        pltpu.SemaphoreType.DMA((SSLOTS,)),
        pltpu.SemaphoreType.DMA,
    ],
)
def _sc_scatter(edge_hbm, si_hbm, st_hbm, out_hbm, acc, ebuf, iring, itail,
                zbuf, lsem, isems, ssem, msem):
    c = lax.axis_index("c")
    s = lax.axis_index("s")
    wid = s * NCORES + c
    base = wid * EPW

    zero = jnp.zeros((16,), jnp.float32)

    @pl.loop(0, ZB)
    def _(r):
        @pl.loop(0, EMB, step=16)
        def _(cc):
            zbuf[r, pl.ds(cc, 16)] = zero

    for k in range(RPS // ZB):
        pltpu.sync_copy(zbuf, acc.at[pl.ds(s * RPS + k * ZB, ZB)])

    @pl.when(s == NSUB - 1)
    def _():
        for t in range(TAIL // ZB):
            pltpu.sync_copy(zbuf, acc.at[pl.ds(NSUB * RPS + t * ZB, ZB)])

    plsc.subcore_barrier()

    def i_cp(j, b):
        return pltpu.make_async_copy(si_hbm.at[wid, j], iring.at[b],
                                     isems.at[b])

    def l_cp(j, b):
        return pltpu.make_async_copy(
            edge_hbm.at[pl.ds(base + j * GW, GW)], ebuf.at[b], lsem.at[b])

    def s_cp(j, b):
        return pltpu.make_async_copy(ebuf.at[b], acc.at[iring.at[b]],
                                     ssem.at[b])

    for b in range(SSLOTS):
        i_cp(b, b).start()
        l_cp(b, b).start()

    def window(j, b):
        i_cp(j, b).wait()
        l_cp(j, b).wait()
        pltpu.async_copy(ebuf.at[b], acc.at[iring.at[b]], ssem.at[b],
                         add=True)
        s_cp(j, b).wait()

        @pl.when(j + SSLOTS < NWIN)
        def _():
            i_cp(j + SSLOTS, b).start()
            l_cp(j + SSLOTS, b).start()

    @pl.loop(0, NWIN // SSLOTS)
    def _(it):
        for b in range(SSLOTS):
            window(it * SSLOTS + b, b)

    for j in range(NWIN - NWIN % SSLOTS, NWIN):
        window(j, j % SSLOTS)

    ct = pltpu.make_async_copy(st_hbm.at[wid], itail, msem)
    ct.start()
    ce = pltpu.make_async_copy(
        edge_hbm.at[pl.ds(base + NWIN * GW, GTAIL)],
        ebuf.at[0].at[pl.ds(0, GTAIL)], msem)
    ce.start()
    ct.wait()
    ce.wait()
    pltpu.sync_copy(ebuf.at[0].at[pl.ds(0, GTAIL)], acc.at[itail.at[0]],
                    add=True)

    plsc.subcore_barrier()
    pltpu.sync_copy(acc.at[pl.ds(s * RPS, RPS)],
                    out_hbm.at[c, pl.ds(s * RPS, RPS)])

    @pl.when(s == NSUB - 1)
    def _():
        pltpu.sync_copy(acc.at[pl.ds(NSUB * RPS, TAIL)],
                        out_hbm.at[c, pl.ds(NSUB * RPS, TAIL)])


def kernel(x, edge_index, edge_attr, emb_table, atom_w, atom_b, ee_w, ee_b,
           ew1, eb1, ew2, eb2, nw1, nb1, nw2, nb2, ow1, ob1, ow2, ob2):
    x2 = x.astype(jnp.int32).reshape(N, 1)
    src = edge_index[0].astype(jnp.int32)
    dst = edge_index[1].astype(jnp.int32)
    src2 = src.reshape(NWORK, EPW)
    dst2 = dst.reshape(NWORK, EPW)
    srcm = src2[:, :NWIN * GW].reshape(NWORK, NWIN, GW)
    srct = src2[:, NWIN * GW:].reshape(NWORK, 1, GTAIL)
    dstm = dst2[:, :NWIN * GW].reshape(NWORK, NWIN, GW)
    dstt = dst2[:, NWIN * GW:].reshape(NWORK, 1, GTAIL)
    d2 = edge_attr.reshape(E, 1)

    We = [ew1[i, :EMB] for i in range(L)]
    Ws = [ew1[i, EMB:2 * EMB] for i in range(L)]
    Wd = [ew1[i, 2 * EMB:] for i in range(L)]

    node, ns, nd = _node_init(
        x2, emb_table, atom_w, atom_b.reshape(1, EMB), Ws[0], Wd[0])
    edge = _edge_init(d2, ee_w, ee_b.reshape(1, EMB))

    for i in range(L):
        gsum = _sc_gather(ns, nd, srcm, dstm, srct, dstt)
        edge = _edge_mlp(edge, gsum, We[i], eb1[i].reshape(1, EMB),
                         ew2[i], eb2[i].reshape(1, EMB))
        parts = _sc_scatter(edge, srcm, srct)
        j = (i + 1) % L
        node, ns, nd = _node_mlp(
            node, parts, nw1[i], nb1[i].reshape(1, EMB), nw2[i],
            nb2[i].reshape(1, EMB), Ws[j], Wd[j])

    out = _readout(node, ow1, ob1.reshape(1, 1), ow2, ob2.reshape(1, 1))
    return out.reshape(1)
